# baseline (speedup 1.0000x reference)
import functools

import jax
import jax.numpy as jnp
from jax import lax
from jax.experimental import pallas as pl
from jax.experimental.pallas import tpu as pltpu
from jax.experimental.pallas import tpu_sc as plsc

N = 10000
E = 320000
G = 100
P = 100
D = 128
D_OUT = 100

NC = 2
NS = 16
NW = NC * NS
EPW = E // NW
CHUNK = 80
NB = 4
EP = 327680
EPP = EP // NW
NCHUNK = EPP // CHUNK
NGRP = NCHUNK // NB
NSENT = 16
ROWS_PT = N // NS
ZROWS = 25
IDXB = 2000
BN_C = 1.0 / (1.0 + 1e-5) ** 0.5

def _mesh():
    return plsc.VectorSubcoreMesh(core_axis_name="c", subcore_axis_name="s",
                                  num_cores=NC, num_subcores=NS)


@functools.cache
def _build_sc_pre():
    return functools.partial(
        pl.kernel,
        out_type=(
            jax.ShapeDtypeStruct((NW, 1, N), jnp.float32),
            jax.ShapeDtypeStruct((NW, G, D), jnp.float32),
            jax.ShapeDtypeStruct((NW, G, 16), jnp.float32),
        ),
        mesh=_mesh(),
        compiler_params=pltpu.CompilerParams(needs_layout_passes=False),
        scratch_types=[
            pltpu.VMEM((N,), jnp.float32),
            pltpu.VMEM((G, D), jnp.float32),
            pltpu.VMEM((G, 16), jnp.float32),
            pltpu.VMEM((IDXB,), jnp.int32),
            pltpu.VMEM((IDXB,), jnp.int32),
        ],
    )(_sc_pre_body)


def _sc_pre_body(src_hbm, dst_hbm, deg_out, flag_out, minn_out,
                 deg_v, flag_v, minn_v, src_b, dst_b):
    c = lax.axis_index("c")
    s = lax.axis_index("s")
    wid = s * NC + c
    base = wid * EPW

    zf = jnp.zeros((16,), jnp.float32)

    def zero_deg(i, _):
        deg_v[pl.ds(i * 16, 16)] = zf
        return ()
    lax.fori_loop(0, N // 16, zero_deg, ())

    def zero_flag(i, _):
        flag_v[i // 8, pl.ds((i % 8) * 16, 16)] = zf
        return ()
    lax.fori_loop(0, G * (D // 16), zero_flag, ())

    def init_minn(i, _):
        minn_v[i, :] = jnp.full((16,), float(N), jnp.float32)
        return ()
    lax.fori_loop(0, G, init_minn, ())

    lane = lax.iota(jnp.int32, 16)
    ones = jnp.ones((16,), jnp.float32)

    def outer(ch, _):
        off = base + ch * IDXB
        pltpu.sync_copy(src_hbm.at[pl.ds(off, IDXB)], src_b)
        pltpu.sync_copy(dst_hbm.at[pl.ds(off, IDXB)], dst_b)

        def inner(j, _):
            src16 = src_b[pl.ds(j * 16, 16)]
            dst16 = dst_b[pl.ds(j * 16, 16)]
            plsc.addupdate_scatter(deg_v, [dst16], ones)
            g_src = src16 // P
            p_src = src16 % P
            g_dst = dst16 // P
            p_dst = dst16 % P
            plsc.store_scatter(flag_v, [g_src, p_src], ones)
            same = g_src == g_dst
            plsc.store_scatter(flag_v, [g_dst, p_dst], ones, mask=same)
            cur = plsc.load_gather(minn_v, [g_src, lane])
            cand = jnp.minimum(src16, dst16).astype(jnp.float32)
            plsc.store_scatter(minn_v, [g_src, lane], jnp.minimum(cur, cand))
            return ()
        lax.fori_loop(0, IDXB // 16, inner, ())
        return ()
    lax.fori_loop(0, EPW // IDXB, outer, ())

    pltpu.sync_copy(deg_v, deg_out.at[wid, 0])
    pltpu.sync_copy(flag_v, flag_out.at[wid])
    pltpu.sync_copy(minn_v, minn_out.at[wid])


@functools.cache
def _build_sc_prop():
    return functools.partial(
        pl.kernel,
        out_type=jax.ShapeDtypeStruct((NC, NS, ROWS_PT, D), jnp.float32),
        mesh=_mesh(),
        compiler_params=pltpu.CompilerParams(needs_layout_passes=False),
        scratch_types=[
            pltpu.VMEM_SHARED((N + NSENT, D), jnp.float32),
            pltpu.VMEM((ZROWS, D), jnp.float32),
        ] + [pltpu.VMEM((CHUNK, D), jnp.float32) for _ in range(NB)]
          + [pltpu.VMEM((CHUNK,), jnp.int32) for _ in range(2 * NB)]
          + [pltpu.SemaphoreType.DMA for _ in range(4 * NB)],
    )(_sc_prop_body)


def _sc_prop_body(ht_hbm, src4, dst4, s_out, acc, zero_v, *bufs):
    rows = bufs[:NB]
    src_i = bufs[NB:2 * NB]
    dst_i = bufs[2 * NB:3 * NB]
    sem_g = bufs[3 * NB:4 * NB]
    sem_s = bufs[4 * NB:5 * NB]
    sem_is = bufs[5 * NB:6 * NB]
    sem_id = bufs[6 * NB:7 * NB]
    c = lax.axis_index("c")
    s = lax.axis_index("s")
    wid = c * NS + s

    zf = jnp.zeros((16,), jnp.float32)

    def zero_buf(i, _):
        zero_v[i // 8, pl.ds((i % 8) * 16, 16)] = zf
        return ()
    lax.fori_loop(0, ZROWS * (D // 16), zero_buf, ())

    def zero_acc(i, _):
        pltpu.sync_copy(zero_v, acc.at[pl.ds(s * ROWS_PT + i * ZROWS, ZROWS)])
        return ()
    lax.fori_loop(0, ROWS_PT // ZROWS, zero_acc, ())

    plsc.subcore_barrier()

    for b in range(NB):
        pltpu.async_copy(src4.at[wid, b, 0], src_i[b], sem_is[b])
        pltpu.async_copy(dst4.at[wid, b, 0], dst_i[b], sem_id[b])

    def outer(k, _):
        for b in range(NB):
            i = k * NB + b

            @pl.when(k > 0)
            def _free_ring():
                pltpu.make_async_copy(rows[b], acc.at[dst_i[b]],
                                      sem_s[b]).wait()
                pltpu.async_copy(dst4.at[wid, i, 0], dst_i[b], sem_id[b])
            pltpu.make_async_copy(src4.at[wid, i, 0], src_i[b],
                                  sem_is[b]).wait()
            pltpu.async_copy(ht_hbm.at[src_i[b]], rows[b], sem_g[b])
        for b in range(NB):
            i = k * NB + b
            pltpu.make_async_copy(ht_hbm.at[src_i[b]], rows[b],
                                  sem_g[b]).wait()

            @pl.when(k < NGRP - 1)
            def _prefetch_src():
                pltpu.async_copy(src4.at[wid, i + NB, 0], src_i[b], sem_is[b])
            pltpu.make_async_copy(dst4.at[wid, i, 0], dst_i[b],
                                  sem_id[b]).wait()
            pltpu.async_copy(rows[b], acc.at[dst_i[b]], sem_s[b], add=True)
        return ()
    lax.fori_loop(0, NGRP, outer, ())

    for b in range(NB):
        pltpu.make_async_copy(rows[b], acc.at[dst_i[b]], sem_s[b]).wait()

    plsc.subcore_barrier()

    pltpu.sync_copy(acc.at[pl.ds(s * ROWS_PT, ROWS_PT)], s_out.at[c, s])


RB = 400
NBLK = N // RB


def _tc0_body(x_ref, w_ref, degp_ref, ht_ref, dinv_ref):
    deg = jnp.sum(degp_ref[...][:, 0, 0, :], axis=0) + 1.0
    dinv = lax.rsqrt(deg)
    y = jnp.dot(x_ref[...], w_ref[...], preferred_element_type=jnp.float32)
    ht_ref[...] = y * dinv[:, None]
    dinv_ref[...] = dinv[:, None]


def _tc0(x, w0, deg_part):
    return pl.pallas_call(
        _tc0_body,
        grid=(NBLK,),
        in_specs=[
            pl.BlockSpec((RB, D), lambda i: (i, 0)),
            pl.BlockSpec((D, D), lambda i: (0, 0)),
            pl.BlockSpec((NW, 1, 1, RB), lambda i: (0, i, 0, 0)),
        ],
        out_specs=[
            pl.BlockSpec((RB, D), lambda i: (i, 0)),
            pl.BlockSpec((RB, 1), lambda i: (i, 0)),
        ],
        out_shape=[
            jax.ShapeDtypeStruct((N, D), jnp.float32),
            jax.ShapeDtypeStruct((N, 1), jnp.float32),
        ],
    )(x, w0, deg_part.reshape(NW, NBLK, 1, RB))


def _tc_mid_body(s_ref, ht_ref, dinv_ref, b_ref, g_ref, be_ref, w_ref, o_ref):
    dinv = dinv_ref[...]
    sm = s_ref[...]
    z = dinv * (sm[0] + sm[1] + ht_ref[...]) + b_ref[...]
    a = jax.nn.relu(z * BN_C * g_ref[...] + be_ref[...])
    y = jnp.dot(a, w_ref[...], preferred_element_type=jnp.float32)
    o_ref[...] = y * dinv


def _tc_mid(s_part, ht, dinv, b, gamma, beta, w_next):
    return pl.pallas_call(
        _tc_mid_body,
        grid=(NBLK,),
        in_specs=[
            pl.BlockSpec((NC, RB, D), lambda i: (0, i, 0)),
            pl.BlockSpec((RB, D), lambda i: (i, 0)),
            pl.BlockSpec((RB, 1), lambda i: (i, 0)),
            pl.BlockSpec((1, D), lambda i: (0, 0)),
            pl.BlockSpec((1, D), lambda i: (0, 0)),
            pl.BlockSpec((1, D), lambda i: (0, 0)),
            pl.BlockSpec((D, D), lambda i: (0, 0)),
        ],
        out_specs=pl.BlockSpec((RB, D), lambda i: (i, 0)),
        out_shape=jax.ShapeDtypeStruct((N, D), jnp.float32),
    )(s_part, ht, dinv, b.reshape(1, D), gamma.reshape(1, D),
      beta.reshape(1, D), w_next)


GB = RB // P


def _tc_fin_body(s_ref, ht_ref, dinv_ref, b_ref, flag_ref, minn_ref, o_ref):
    i = pl.program_id(0)
    sm = s_ref[...]
    h3 = dinv_ref[...] * (sm[0] + sm[1] + ht_ref[...]) + b_ref[...]
    ga = lax.broadcasted_iota(jnp.int32, (GB, RB), 0)
    ra = lax.broadcasted_iota(jnp.int32, (GB, RB), 1) // P
    sel = jnp.where(ga == ra, 1.0 / P, 0.0).astype(jnp.float32)
    pooled = jnp.dot(sel, h3, preferred_element_type=jnp.float32)
    flg = jnp.max(flag_ref[...][:, 0], axis=0)
    mn = jnp.min(jnp.min(minn_ref[...][:, 0], axis=2), axis=0)
    aa = lax.broadcasted_iota(jnp.int32, (GB, D), 0)
    jj = lax.broadcasted_iota(jnp.int32, (GB, D), 1)
    nid = ((i * GB + aa) * P + jj).astype(jnp.float32)
    mask = (flg > 0.5) & (nid != mn[:, None])
    outv = jnp.where(mask, jnp.float32(-1e10), pooled)
    o_ref[...] = outv[None, :, :D_OUT]


def _tc_fin(s_part, ht, dinv, b2p, flag_part, minn_part):
    return pl.pallas_call(
        _tc_fin_body,
        grid=(NBLK,),
        in_specs=[
            pl.BlockSpec((NC, RB, D), lambda i: (0, i, 0)),
            pl.BlockSpec((RB, D), lambda i: (i, 0)),
            pl.BlockSpec((RB, 1), lambda i: (i, 0)),
            pl.BlockSpec((1, D), lambda i: (0, 0)),
            pl.BlockSpec((NW, 1, GB, D), lambda i: (0, i, 0, 0)),
            pl.BlockSpec((NW, 1, GB, 16), lambda i: (0, i, 0, 0)),
        ],
        out_specs=pl.BlockSpec((1, GB, D_OUT), lambda i: (i, 0, 0)),
        out_shape=jax.ShapeDtypeStruct((NBLK, GB, D_OUT), jnp.float32),
    )(s_part, ht, dinv, b2p.reshape(1, D),
      flag_part.reshape(NW, NBLK, GB, D),
      minn_part.reshape(NW, NBLK, GB, 16)).reshape(G, D_OUT)


def kernel(x, edge_index, batch, W0, b0, gamma0, beta0,
           W1, b1, gamma1, beta1, W2, b2):
    src = edge_index[0]
    dst = edge_index[1]

    sc_pre = _build_sc_pre()
    sc_prop = _build_sc_prop()
    deg_part, flag_part, minn_part = sc_pre(src, dst)
    deg_part = deg_part.reshape(NW, N)
    npad = EP - E
    srcp = jnp.concatenate([src, jnp.zeros((npad,), jnp.int32)])
    dstp = jnp.concatenate(
        [dst, N + (jnp.arange(npad, dtype=jnp.int32) % NSENT)])
    src2d = srcp.reshape(NW, NCHUNK, 1, CHUNK)
    dst2d = dstp.reshape(NW, NCHUNK, 1, CHUNK)

    ht0, dinv = _tc0(x, W0, deg_part)
    s0 = sc_prop(ht0, src2d, dst2d).reshape(NC, N, D)
    ht1 = _tc_mid(s0, ht0, dinv, b0, gamma0, beta0, W1)
    s1 = sc_prop(ht1, src2d, dst2d).reshape(NC, N, D)
    w2p = jnp.pad(W2, ((0, 0), (0, D - D_OUT)))
    ht2 = _tc_mid(s1, ht1, dinv, b1, gamma1, beta1, w2p)
    s2 = sc_prop(ht2, src2d, dst2d).reshape(NC, N, D)
    b2p = jnp.pad(b2, (0, D - D_OUT))
    return _tc_fin(s2, ht2, dinv, b2p, flag_part, minn_part)


# speedup vs baseline: 1.2497x; 1.2497x over previous
import functools

import jax
import jax.numpy as jnp
from jax import lax
from jax.experimental import pallas as pl
from jax.experimental.pallas import tpu as pltpu
from jax.experimental.pallas import tpu_sc as plsc

N = 10000
E = 320000
G = 100
P = 100
D = 128
D_OUT = 100

NC = 2
NS = 16
NW = NC * NS
EPW = E // NW
CHUNK = 80
NB = 4
EP = 327680
EPP = EP // NW
NCHUNK = EPP // CHUNK
NGRP = NCHUNK // NB
NSENT = 256
ROWS_PT = N // NS
ZROWS = 25
IDXB = 2000
BN_C = 1.0 / (1.0 + 1e-5) ** 0.5

def _mesh():
    return plsc.VectorSubcoreMesh(core_axis_name="c", subcore_axis_name="s",
                                  num_cores=NC, num_subcores=NS)


@functools.cache
def _build_sc_pre():
    return functools.partial(
        pl.kernel,
        out_type=(
            jax.ShapeDtypeStruct((NW, 1, N), jnp.float32),
            jax.ShapeDtypeStruct((NW, G, D), jnp.float32),
            jax.ShapeDtypeStruct((NW, G, 16), jnp.float32),
        ),
        mesh=_mesh(),
        compiler_params=pltpu.CompilerParams(needs_layout_passes=False),
        scratch_types=[
            pltpu.VMEM((N,), jnp.float32),
            pltpu.VMEM((G, D), jnp.float32),
            pltpu.VMEM((G, 16), jnp.float32),
            pltpu.VMEM((IDXB,), jnp.int32),
            pltpu.VMEM((IDXB,), jnp.int32),
        ],
    )(_sc_pre_body)


def _sc_pre_body(src_hbm, dst_hbm, deg_out, flag_out, minn_out,
                 deg_v, flag_v, minn_v, src_b, dst_b):
    c = lax.axis_index("c")
    s = lax.axis_index("s")
    wid = s * NC + c
    base = wid * EPW

    zf = jnp.zeros((16,), jnp.float32)

    def zero_deg(i, _):
        deg_v[pl.ds(i * 16, 16)] = zf
        return ()
    lax.fori_loop(0, N // 16, zero_deg, ())

    def zero_flag(i, _):
        flag_v[i // 8, pl.ds((i % 8) * 16, 16)] = zf
        return ()
    lax.fori_loop(0, G * (D // 16), zero_flag, ())

    def init_minn(i, _):
        minn_v[i, :] = jnp.full((16,), float(N), jnp.float32)
        return ()
    lax.fori_loop(0, G, init_minn, ())

    lane = lax.iota(jnp.int32, 16)
    ones = jnp.ones((16,), jnp.float32)

    def outer(ch, _):
        off = base + ch * IDXB
        pltpu.sync_copy(src_hbm.at[pl.ds(off, IDXB)], src_b)
        pltpu.sync_copy(dst_hbm.at[pl.ds(off, IDXB)], dst_b)

        def inner(j, _):
            src16 = src_b[pl.ds(j * 16, 16)]
            dst16 = dst_b[pl.ds(j * 16, 16)]
            plsc.addupdate_scatter(deg_v, [dst16], ones)
            g_src = src16 // P
            p_src = src16 % P
            g_dst = dst16 // P
            p_dst = dst16 % P
            plsc.store_scatter(flag_v, [g_src, p_src], ones)
            same = g_src == g_dst
            plsc.store_scatter(flag_v, [g_dst, p_dst], ones, mask=same)
            cur = plsc.load_gather(minn_v, [g_src, lane])
            cand = jnp.minimum(src16, dst16).astype(jnp.float32)
            plsc.store_scatter(minn_v, [g_src, lane], jnp.minimum(cur, cand))
            return ()
        lax.fori_loop(0, IDXB // 16, inner, ())
        return ()
    lax.fori_loop(0, EPW // IDXB, outer, ())

    pltpu.sync_copy(deg_v, deg_out.at[wid, 0])
    pltpu.sync_copy(flag_v, flag_out.at[wid])
    pltpu.sync_copy(minn_v, minn_out.at[wid])


@functools.cache
def _build_sc_prop():
    return functools.partial(
        pl.kernel,
        out_type=jax.ShapeDtypeStruct((NC, NS, ROWS_PT, D), jnp.float32),
        mesh=_mesh(),
        compiler_params=pltpu.CompilerParams(needs_layout_passes=False),
        scratch_types=[
            pltpu.VMEM_SHARED((N + NSENT, D), jnp.float32),
            pltpu.VMEM((ZROWS, D), jnp.float32),
        ] + [pltpu.VMEM((CHUNK, D), jnp.float32) for _ in range(NB)]
          + [pltpu.VMEM((CHUNK,), jnp.int32) for _ in range(2 * NB)]
          + [pltpu.SemaphoreType.DMA for _ in range(4 * NB)],
    )(_sc_prop_body)


def _sc_prop_body(ht_hbm, src4, dst4, s_out, acc, zero_v, *bufs):
    rows = bufs[:NB]
    src_i = bufs[NB:2 * NB]
    dst_i = bufs[2 * NB:3 * NB]
    sem_g = bufs[3 * NB:4 * NB]
    sem_s = bufs[4 * NB:5 * NB]
    sem_is = bufs[5 * NB:6 * NB]
    sem_id = bufs[6 * NB:7 * NB]
    c = lax.axis_index("c")
    s = lax.axis_index("s")
    wid = c * NS + s

    zf = jnp.zeros((16,), jnp.float32)

    def zero_buf(i, _):
        zero_v[i // 8, pl.ds((i % 8) * 16, 16)] = zf
        return ()
    lax.fori_loop(0, ZROWS * (D // 16), zero_buf, ())

    def zero_acc(i, _):
        pltpu.sync_copy(zero_v, acc.at[pl.ds(s * ROWS_PT + i * ZROWS, ZROWS)])
        return ()
    lax.fori_loop(0, ROWS_PT // ZROWS, zero_acc, ())

    plsc.subcore_barrier()

    for b in range(NB):
        pltpu.async_copy(src4.at[wid, b, 0], src_i[b], sem_is[b])
        pltpu.async_copy(dst4.at[wid, b, 0], dst_i[b], sem_id[b])

    def outer(k, _):
        for b in range(NB):
            i = k * NB + b

            @pl.when(k > 0)
            def _free_ring():
                pltpu.make_async_copy(rows[b], acc.at[dst_i[b]],
                                      sem_s[b]).wait()
                pltpu.async_copy(dst4.at[wid, i, 0], dst_i[b], sem_id[b])
            pltpu.make_async_copy(src4.at[wid, i, 0], src_i[b],
                                  sem_is[b]).wait()
            pltpu.async_copy(ht_hbm.at[src_i[b]], rows[b], sem_g[b])
        for b in range(NB):
            i = k * NB + b
            pltpu.make_async_copy(ht_hbm.at[src_i[b]], rows[b],
                                  sem_g[b]).wait()

            @pl.when(k < NGRP - 1)
            def _prefetch_src():
                pltpu.async_copy(src4.at[wid, i + NB, 0], src_i[b], sem_is[b])
            pltpu.make_async_copy(dst4.at[wid, i, 0], dst_i[b],
                                  sem_id[b]).wait()
            pltpu.async_copy(rows[b], acc.at[dst_i[b]], sem_s[b], add=True)
        return ()
    lax.fori_loop(0, NGRP, outer, ())

    for b in range(NB):
        pltpu.make_async_copy(rows[b], acc.at[dst_i[b]], sem_s[b]).wait()

    plsc.subcore_barrier()

    pltpu.sync_copy(acc.at[pl.ds(s * ROWS_PT, ROWS_PT)], s_out.at[c, s])


RB = 400
NBLK = N // RB


def _tc0_body(x_ref, w_ref, degp_ref, ht_ref, dinv_ref):
    deg = jnp.sum(degp_ref[...][:, 0, 0, :], axis=0) + 1.0
    dinv = lax.rsqrt(deg)
    y = jnp.dot(x_ref[...], w_ref[...], preferred_element_type=jnp.float32)
    ht_ref[...] = y * dinv[:, None]
    dinv_ref[...] = dinv[:, None]


def _tc0(x, w0, deg_part):
    return pl.pallas_call(
        _tc0_body,
        grid=(NBLK,),
        in_specs=[
            pl.BlockSpec((RB, D), lambda i: (i, 0)),
            pl.BlockSpec((D, D), lambda i: (0, 0)),
            pl.BlockSpec((NW, 1, 1, RB), lambda i: (0, i, 0, 0)),
        ],
        out_specs=[
            pl.BlockSpec((RB, D), lambda i: (i, 0)),
            pl.BlockSpec((RB, 1), lambda i: (i, 0)),
        ],
        out_shape=[
            jax.ShapeDtypeStruct((N, D), jnp.float32),
            jax.ShapeDtypeStruct((N, 1), jnp.float32),
        ],
    )(x, w0, deg_part.reshape(NW, NBLK, 1, RB))


def _tc_mid_body(s_ref, ht_ref, dinv_ref, b_ref, g_ref, be_ref, w_ref, o_ref):
    dinv = dinv_ref[...]
    sm = s_ref[...]
    z = dinv * (sm[0] + sm[1] + ht_ref[...]) + b_ref[...]
    a = jax.nn.relu(z * BN_C * g_ref[...] + be_ref[...])
    y = jnp.dot(a, w_ref[...], preferred_element_type=jnp.float32)
    o_ref[...] = y * dinv


def _tc_mid(s_part, ht, dinv, b, gamma, beta, w_next):
    return pl.pallas_call(
        _tc_mid_body,
        grid=(NBLK,),
        in_specs=[
            pl.BlockSpec((NC, RB, D), lambda i: (0, i, 0)),
            pl.BlockSpec((RB, D), lambda i: (i, 0)),
            pl.BlockSpec((RB, 1), lambda i: (i, 0)),
            pl.BlockSpec((1, D), lambda i: (0, 0)),
            pl.BlockSpec((1, D), lambda i: (0, 0)),
            pl.BlockSpec((1, D), lambda i: (0, 0)),
            pl.BlockSpec((D, D), lambda i: (0, 0)),
        ],
        out_specs=pl.BlockSpec((RB, D), lambda i: (i, 0)),
        out_shape=jax.ShapeDtypeStruct((N, D), jnp.float32),
    )(s_part, ht, dinv, b.reshape(1, D), gamma.reshape(1, D),
      beta.reshape(1, D), w_next)


GB = RB // P


def _tc_fin_body(s_ref, ht_ref, dinv_ref, b_ref, flag_ref, minn_ref, o_ref):
    i = pl.program_id(0)
    sm = s_ref[...]
    h3 = dinv_ref[...] * (sm[0] + sm[1] + ht_ref[...]) + b_ref[...]
    ga = lax.broadcasted_iota(jnp.int32, (GB, RB), 0)
    ra = lax.broadcasted_iota(jnp.int32, (GB, RB), 1) // P
    sel = jnp.where(ga == ra, 1.0 / P, 0.0).astype(jnp.float32)
    pooled = jnp.dot(sel, h3, preferred_element_type=jnp.float32)
    flg = jnp.max(flag_ref[...][:, 0], axis=0)
    mn = jnp.min(jnp.min(minn_ref[...][:, 0], axis=2), axis=0)
    aa = lax.broadcasted_iota(jnp.int32, (GB, D), 0)
    jj = lax.broadcasted_iota(jnp.int32, (GB, D), 1)
    nid = ((i * GB + aa) * P + jj).astype(jnp.float32)
    mask = (flg > 0.5) & (nid != mn[:, None])
    outv = jnp.where(mask, jnp.float32(-1e10), pooled)
    o_ref[...] = outv[None, :, :D_OUT]


def _tc_fin(s_part, ht, dinv, b2p, flag_part, minn_part):
    return pl.pallas_call(
        _tc_fin_body,
        grid=(NBLK,),
        in_specs=[
            pl.BlockSpec((NC, RB, D), lambda i: (0, i, 0)),
            pl.BlockSpec((RB, D), lambda i: (i, 0)),
            pl.BlockSpec((RB, 1), lambda i: (i, 0)),
            pl.BlockSpec((1, D), lambda i: (0, 0)),
            pl.BlockSpec((NW, 1, GB, D), lambda i: (0, i, 0, 0)),
            pl.BlockSpec((NW, 1, GB, 16), lambda i: (0, i, 0, 0)),
        ],
        out_specs=pl.BlockSpec((1, GB, D_OUT), lambda i: (i, 0, 0)),
        out_shape=jax.ShapeDtypeStruct((NBLK, GB, D_OUT), jnp.float32),
    )(s_part, ht, dinv, b2p.reshape(1, D),
      flag_part.reshape(NW, NBLK, GB, D),
      minn_part.reshape(NW, NBLK, GB, 16)).reshape(G, D_OUT)


def kernel(x, edge_index, batch, W0, b0, gamma0, beta0,
           W1, b1, gamma1, beta1, W2, b2):
    src = edge_index[0]
    dst = edge_index[1]

    sc_pre = _build_sc_pre()
    sc_prop = _build_sc_prop()
    deg_part, flag_part, minn_part = sc_pre(src, dst)
    deg_part = deg_part.reshape(NW, N)
    fpw = EPP - EPW
    srcp = jnp.concatenate(
        [src.reshape(NW, EPW), jnp.zeros((NW, fpw), jnp.int32)], axis=1)
    fake_dst = N + (jnp.arange(fpw, dtype=jnp.int32) % NSENT)
    dstp = jnp.concatenate(
        [dst.reshape(NW, EPW), jnp.broadcast_to(fake_dst, (NW, fpw))], axis=1)
    src2d = srcp.reshape(NW, NCHUNK, 1, CHUNK)
    dst2d = dstp.reshape(NW, NCHUNK, 1, CHUNK)

    ht0, dinv = _tc0(x, W0, deg_part)
    s0 = sc_prop(ht0, src2d, dst2d).reshape(NC, N, D)
    ht1 = _tc_mid(s0, ht0, dinv, b0, gamma0, beta0, W1)
    s1 = sc_prop(ht1, src2d, dst2d).reshape(NC, N, D)
    w2p = jnp.pad(W2, ((0, 0), (0, D - D_OUT)))
    ht2 = _tc_mid(s1, ht1, dinv, b1, gamma1, beta1, w2p)
    s2 = sc_prop(ht2, src2d, dst2d).reshape(NC, N, D)
    b2p = jnp.pad(b2, (0, D - D_OUT))
    return _tc_fin(s2, ht2, dinv, b2p, flag_part, minn_part)


# speedup vs baseline: 3.1905x; 2.5530x over previous
import functools

import jax
import jax.numpy as jnp
from jax import lax
from jax.experimental import pallas as pl
from jax.experimental.pallas import tpu as pltpu
from jax.experimental.pallas import tpu_sc as plsc

N = 10000
E = 320000
G = 100
P = 100
D = 128
D_OUT = 100

NC = 2
NS = 16
NW = NC * NS
EPW = E // NW
CHUNK = 80
NB = 4
NCHUNK = EPW // CHUNK
NGRP = (NCHUNK - 1) // NB
ROWS_PT = N // NS
ZROWS = 25
IDXB = 2000
BN_C = 1.0 / (1.0 + 1e-5) ** 0.5

def _mesh():
    return plsc.VectorSubcoreMesh(core_axis_name="c", subcore_axis_name="s",
                                  num_cores=NC, num_subcores=NS)


@functools.cache
def _build_sc_pre():
    return functools.partial(
        pl.kernel,
        out_type=(
            jax.ShapeDtypeStruct((NW, 1, N), jnp.float32),
            jax.ShapeDtypeStruct((NW, G, D), jnp.float32),
            jax.ShapeDtypeStruct((NW, G, 16), jnp.float32),
        ),
        mesh=_mesh(),
        compiler_params=pltpu.CompilerParams(needs_layout_passes=False),
        scratch_types=[
            pltpu.VMEM((N,), jnp.float32),
            pltpu.VMEM((G, D), jnp.float32),
            pltpu.VMEM((G, 16), jnp.float32),
            pltpu.VMEM((IDXB,), jnp.int32),
            pltpu.VMEM((IDXB,), jnp.int32),
        ],
    )(_sc_pre_body)


def _sc_pre_body(src_hbm, dst_hbm, deg_out, flag_out, minn_out,
                 deg_v, flag_v, minn_v, src_b, dst_b):
    c = lax.axis_index("c")
    s = lax.axis_index("s")
    wid = s * NC + c
    base = wid * EPW

    zf = jnp.zeros((16,), jnp.float32)

    def zero_deg(i, _):
        deg_v[pl.ds(i * 16, 16)] = zf
        return ()
    lax.fori_loop(0, N // 16, zero_deg, ())

    def zero_flag(i, _):
        flag_v[i // 8, pl.ds((i % 8) * 16, 16)] = zf
        return ()
    lax.fori_loop(0, G * (D // 16), zero_flag, ())

    def init_minn(i, _):
        minn_v[i, :] = jnp.full((16,), float(N), jnp.float32)
        return ()
    lax.fori_loop(0, G, init_minn, ())

    lane = lax.iota(jnp.int32, 16)
    ones = jnp.ones((16,), jnp.float32)

    def outer(ch, _):
        off = base + ch * IDXB
        pltpu.sync_copy(src_hbm.at[pl.ds(off, IDXB)], src_b)
        pltpu.sync_copy(dst_hbm.at[pl.ds(off, IDXB)], dst_b)

        def inner(j, _):
            src16 = src_b[pl.ds(j * 16, 16)]
            dst16 = dst_b[pl.ds(j * 16, 16)]
            plsc.addupdate_scatter(deg_v, [dst16], ones)
            g_src = src16 // P
            p_src = src16 % P
            g_dst = dst16 // P
            p_dst = dst16 % P
            plsc.store_scatter(flag_v, [g_src, p_src], ones)
            same = g_src == g_dst
            plsc.store_scatter(flag_v, [g_dst, p_dst], ones, mask=same)
            cur = plsc.load_gather(minn_v, [g_src, lane])
            cand = jnp.minimum(src16, dst16).astype(jnp.float32)
            plsc.store_scatter(minn_v, [g_src, lane], jnp.minimum(cur, cand))
            return ()
        lax.fori_loop(0, IDXB // 16, inner, ())
        return ()
    lax.fori_loop(0, EPW // IDXB, outer, ())

    pltpu.sync_copy(deg_v, deg_out.at[wid, 0])
    pltpu.sync_copy(flag_v, flag_out.at[wid])
    pltpu.sync_copy(minn_v, minn_out.at[wid])


@functools.cache
def _build_sc_prop():
    return functools.partial(
        pl.kernel,
        out_type=jax.ShapeDtypeStruct((NC, NS, ROWS_PT, D), jnp.float32),
        mesh=_mesh(),
        compiler_params=pltpu.CompilerParams(needs_layout_passes=False),
        scratch_types=[
            pltpu.VMEM_SHARED((N, D), jnp.float32),
            pltpu.VMEM((ZROWS, D), jnp.float32),
        ] + [pltpu.VMEM((CHUNK, D), jnp.float32) for _ in range(NB)]
          + [pltpu.VMEM((CHUNK,), jnp.int32) for _ in range(2 * NB)]
          + [pltpu.SemaphoreType.DMA for _ in range(4 * NB)],
    )(_sc_prop_body)


def _sc_prop_body(ht_hbm, src4, dst4, s_out, acc, zero_v, *bufs):
    rows = bufs[:NB]
    src_i = bufs[NB:2 * NB]
    dst_i = bufs[2 * NB:3 * NB]
    sem_g = bufs[3 * NB:4 * NB]
    sem_s = bufs[4 * NB:5 * NB]
    sem_is = bufs[5 * NB:6 * NB]
    sem_id = bufs[6 * NB:7 * NB]
    c = lax.axis_index("c")
    s = lax.axis_index("s")
    wid = c * NS + s

    zf = jnp.zeros((16,), jnp.float32)

    def zero_buf(i, _):
        zero_v[i // 8, pl.ds((i % 8) * 16, 16)] = zf
        return ()
    lax.fori_loop(0, ZROWS * (D // 16), zero_buf, ())

    def zero_acc(i, _):
        pltpu.sync_copy(zero_v, acc.at[pl.ds(s * ROWS_PT + i * ZROWS, ZROWS)])
        return ()
    lax.fori_loop(0, ROWS_PT // ZROWS, zero_acc, ())

    plsc.subcore_barrier()

    for b in range(NB):
        pltpu.async_copy(src4.at[wid, b, 0], src_i[b], sem_is[b])
        pltpu.async_copy(dst4.at[wid, b, 0], dst_i[b], sem_id[b])

    def outer(k, _):
        for b in range(NB):
            i = k * NB + b

            @pl.when(k > 0)
            def _free_ring():
                pltpu.make_async_copy(rows[b], acc.at[dst_i[b]],
                                      sem_s[b]).wait()
                pltpu.async_copy(dst4.at[wid, i, 0], dst_i[b], sem_id[b])
            pltpu.make_async_copy(src4.at[wid, i, 0], src_i[b],
                                  sem_is[b]).wait()
            pltpu.async_copy(ht_hbm.at[src_i[b]], rows[b], sem_g[b])
        for b in range(NB):
            i = k * NB + b
            pltpu.make_async_copy(ht_hbm.at[src_i[b]], rows[b],
                                  sem_g[b]).wait()

            if b == 0:
                pltpu.async_copy(src4.at[wid, i + NB, 0], src_i[b], sem_is[b])
            else:
                @pl.when(k < NGRP - 1)
                def _prefetch_src():
                    pltpu.async_copy(src4.at[wid, i + NB, 0], src_i[b],
                                     sem_is[b])
            pltpu.make_async_copy(dst4.at[wid, i, 0], dst_i[b],
                                  sem_id[b]).wait()
            pltpu.async_copy(rows[b], acc.at[dst_i[b]], sem_s[b], add=True)
        return ()
    lax.fori_loop(0, NGRP, outer, ())

    tail = NCHUNK - 1
    pltpu.make_async_copy(rows[0], acc.at[dst_i[0]], sem_s[0]).wait()
    pltpu.async_copy(dst4.at[wid, tail, 0], dst_i[0], sem_id[0])
    pltpu.make_async_copy(src4.at[wid, tail, 0], src_i[0], sem_is[0]).wait()
    pltpu.async_copy(ht_hbm.at[src_i[0]], rows[0], sem_g[0])
    pltpu.make_async_copy(ht_hbm.at[src_i[0]], rows[0], sem_g[0]).wait()
    pltpu.make_async_copy(dst4.at[wid, tail, 0], dst_i[0], sem_id[0]).wait()
    pltpu.async_copy(rows[0], acc.at[dst_i[0]], sem_s[0], add=True)

    for b in range(NB):
        pltpu.make_async_copy(rows[b], acc.at[dst_i[b]], sem_s[b]).wait()

    plsc.subcore_barrier()

    pltpu.sync_copy(acc.at[pl.ds(s * ROWS_PT, ROWS_PT)], s_out.at[c, s])


RB = 400
NBLK = N // RB


def _tc0_body(x_ref, w_ref, degp_ref, ht_ref, dinv_ref):
    deg = jnp.sum(degp_ref[...][:, 0, 0, :], axis=0) + 1.0
    dinv = lax.rsqrt(deg)
    y = jnp.dot(x_ref[...], w_ref[...], preferred_element_type=jnp.float32)
    ht_ref[...] = y * dinv[:, None]
    dinv_ref[...] = dinv[:, None]


def _tc0(x, w0, deg_part):
    return pl.pallas_call(
        _tc0_body,
        grid=(NBLK,),
        in_specs=[
            pl.BlockSpec((RB, D), lambda i: (i, 0)),
            pl.BlockSpec((D, D), lambda i: (0, 0)),
            pl.BlockSpec((NW, 1, 1, RB), lambda i: (0, i, 0, 0)),
        ],
        out_specs=[
            pl.BlockSpec((RB, D), lambda i: (i, 0)),
            pl.BlockSpec((RB, 1), lambda i: (i, 0)),
        ],
        out_shape=[
            jax.ShapeDtypeStruct((N, D), jnp.float32),
            jax.ShapeDtypeStruct((N, 1), jnp.float32),
        ],
    )(x, w0, deg_part.reshape(NW, NBLK, 1, RB))


def _tc_mid_body(s_ref, ht_ref, dinv_ref, b_ref, g_ref, be_ref, w_ref, o_ref):
    dinv = dinv_ref[...]
    sm = s_ref[...]
    z = dinv * (sm[0] + sm[1] + ht_ref[...]) + b_ref[...]
    a = jax.nn.relu(z * BN_C * g_ref[...] + be_ref[...])
    y = jnp.dot(a, w_ref[...], preferred_element_type=jnp.float32)
    o_ref[...] = y * dinv


def _tc_mid(s_part, ht, dinv, b, gamma, beta, w_next):
    return pl.pallas_call(
        _tc_mid_body,
        grid=(NBLK,),
        in_specs=[
            pl.BlockSpec((NC, RB, D), lambda i: (0, i, 0)),
            pl.BlockSpec((RB, D), lambda i: (i, 0)),
            pl.BlockSpec((RB, 1), lambda i: (i, 0)),
            pl.BlockSpec((1, D), lambda i: (0, 0)),
            pl.BlockSpec((1, D), lambda i: (0, 0)),
            pl.BlockSpec((1, D), lambda i: (0, 0)),
            pl.BlockSpec((D, D), lambda i: (0, 0)),
        ],
        out_specs=pl.BlockSpec((RB, D), lambda i: (i, 0)),
        out_shape=jax.ShapeDtypeStruct((N, D), jnp.float32),
    )(s_part, ht, dinv, b.reshape(1, D), gamma.reshape(1, D),
      beta.reshape(1, D), w_next)


GB = RB // P


def _tc_fin_body(s_ref, ht_ref, dinv_ref, b_ref, flag_ref, minn_ref, o_ref):
    i = pl.program_id(0)
    sm = s_ref[...]
    h3 = dinv_ref[...] * (sm[0] + sm[1] + ht_ref[...]) + b_ref[...]
    ga = lax.broadcasted_iota(jnp.int32, (GB, RB), 0)
    ra = lax.broadcasted_iota(jnp.int32, (GB, RB), 1) // P
    sel = jnp.where(ga == ra, 1.0 / P, 0.0).astype(jnp.float32)
    pooled = jnp.dot(sel, h3, preferred_element_type=jnp.float32)
    flg = jnp.max(flag_ref[...][:, 0], axis=0)
    mn = jnp.min(jnp.min(minn_ref[...][:, 0], axis=2), axis=0)
    aa = lax.broadcasted_iota(jnp.int32, (GB, D), 0)
    jj = lax.broadcasted_iota(jnp.int32, (GB, D), 1)
    nid = ((i * GB + aa) * P + jj).astype(jnp.float32)
    mask = (flg > 0.5) & (nid != mn[:, None])
    outv = jnp.where(mask, jnp.float32(-1e10), pooled)
    o_ref[...] = outv[None, :, :D_OUT]


def _tc_fin(s_part, ht, dinv, b2p, flag_part, minn_part):
    return pl.pallas_call(
        _tc_fin_body,
        grid=(NBLK,),
        in_specs=[
            pl.BlockSpec((NC, RB, D), lambda i: (0, i, 0)),
            pl.BlockSpec((RB, D), lambda i: (i, 0)),
            pl.BlockSpec((RB, 1), lambda i: (i, 0)),
            pl.BlockSpec((1, D), lambda i: (0, 0)),
            pl.BlockSpec((NW, 1, GB, D), lambda i: (0, i, 0, 0)),
            pl.BlockSpec((NW, 1, GB, 16), lambda i: (0, i, 0, 0)),
        ],
        out_specs=pl.BlockSpec((1, GB, D_OUT), lambda i: (i, 0, 0)),
        out_shape=jax.ShapeDtypeStruct((NBLK, GB, D_OUT), jnp.float32),
    )(s_part, ht, dinv, b2p.reshape(1, D),
      flag_part.reshape(NW, NBLK, GB, D),
      minn_part.reshape(NW, NBLK, GB, 16)).reshape(G, D_OUT)


def kernel(x, edge_index, batch, W0, b0, gamma0, beta0,
           W1, b1, gamma1, beta1, W2, b2):
    src = edge_index[0]
    dst = edge_index[1]

    sc_pre = _build_sc_pre()
    sc_prop = _build_sc_prop()
    deg_part, flag_part, minn_part = sc_pre(src, dst)
    deg_part = deg_part.reshape(NW, N)
    src2d = src.reshape(NW, NCHUNK, 1, CHUNK)
    dst2d = dst.reshape(NW, NCHUNK, 1, CHUNK)

    ht0, dinv = _tc0(x, W0, deg_part)
    s0 = sc_prop(ht0, src2d, dst2d).reshape(NC, N, D)
    ht1 = _tc_mid(s0, ht0, dinv, b0, gamma0, beta0, W1)
    s1 = sc_prop(ht1, src2d, dst2d).reshape(NC, N, D)
    w2p = jnp.pad(W2, ((0, 0), (0, D - D_OUT)))
    ht2 = _tc_mid(s1, ht1, dinv, b1, gamma1, beta1, w2p)
    s2 = sc_prop(ht2, src2d, dst2d).reshape(NC, N, D)
    b2p = jnp.pad(b2, (0, D - D_OUT))
    return _tc_fin(s2, ht2, dinv, b2p, flag_part, minn_part)


# speedup vs baseline: 3.1956x; 1.0016x over previous
import functools

import jax
import jax.numpy as jnp
from jax import lax
from jax.experimental import pallas as pl
from jax.experimental.pallas import tpu as pltpu
from jax.experimental.pallas import tpu_sc as plsc

N = 10000
E = 320000
G = 100
P = 100
D = 128
D_OUT = 100

NC = 2
NS = 16
NW = NC * NS
EPW = E // NW
CHUNK = 80
NB = 4
NCHUNK = EPW // CHUNK
NGRP = (NCHUNK - 1) // NB
ROWS_PT = N // NS
ZROWS = 25
IDXB = 2000
BN_C = 1.0 / (1.0 + 1e-5) ** 0.5

def _mesh():
    return plsc.VectorSubcoreMesh(core_axis_name="c", subcore_axis_name="s",
                                  num_cores=NC, num_subcores=NS)


@functools.cache
def _build_sc_pre():
    return functools.partial(
        pl.kernel,
        out_type=(
            jax.ShapeDtypeStruct((NW, 1, N), jnp.float32),
            jax.ShapeDtypeStruct((NW, G, D), jnp.float32),
            jax.ShapeDtypeStruct((NW, G, 16), jnp.float32),
        ),
        mesh=_mesh(),
        compiler_params=pltpu.CompilerParams(needs_layout_passes=False),
        scratch_types=[
            pltpu.VMEM((N,), jnp.float32),
            pltpu.VMEM((G, D), jnp.float32),
            pltpu.VMEM((G, 16), jnp.float32),
            pltpu.VMEM((IDXB,), jnp.int32),
            pltpu.VMEM((IDXB,), jnp.int32),
        ],
    )(_sc_pre_body)


def _sc_pre_body(src_hbm, dst_hbm, deg_out, flag_out, minn_out,
                 deg_v, flag_v, minn_v, src_b, dst_b):
    c = lax.axis_index("c")
    s = lax.axis_index("s")
    wid = s * NC + c
    base = wid * EPW

    zf = jnp.zeros((16,), jnp.float32)

    def zero_deg(i, _):
        deg_v[pl.ds(i * 16, 16)] = zf
        return ()
    lax.fori_loop(0, N // 16, zero_deg, ())

    def zero_flag(i, _):
        flag_v[i // 8, pl.ds((i % 8) * 16, 16)] = zf
        return ()
    lax.fori_loop(0, G * (D // 16), zero_flag, ())

    def init_minn(i, _):
        minn_v[i, :] = jnp.full((16,), float(N), jnp.float32)
        return ()
    lax.fori_loop(0, G, init_minn, ())

    lane = lax.iota(jnp.int32, 16)
    ones = jnp.ones((16,), jnp.float32)

    def outer(ch, _):
        off = base + ch * IDXB
        pltpu.sync_copy(src_hbm.at[pl.ds(off, IDXB)], src_b)
        pltpu.sync_copy(dst_hbm.at[pl.ds(off, IDXB)], dst_b)

        def inner(j, _):
            src16 = src_b[pl.ds(j * 16, 16)]
            dst16 = dst_b[pl.ds(j * 16, 16)]
            plsc.addupdate_scatter(deg_v, [dst16], ones)
            g_src = src16 // P
            p_src = src16 % P
            g_dst = dst16 // P
            p_dst = dst16 % P
            plsc.store_scatter(flag_v, [g_src, p_src], ones)
            same = g_src == g_dst
            plsc.store_scatter(flag_v, [g_dst, p_dst], ones, mask=same)
            cur = plsc.load_gather(minn_v, [g_src, lane])
            cand = jnp.minimum(src16, dst16).astype(jnp.float32)
            plsc.store_scatter(minn_v, [g_src, lane], jnp.minimum(cur, cand))
            return ()
        lax.fori_loop(0, IDXB // 16, inner, ())
        return ()
    lax.fori_loop(0, EPW // IDXB, outer, ())

    pltpu.sync_copy(deg_v, deg_out.at[wid, 0])
    pltpu.sync_copy(flag_v, flag_out.at[wid])
    pltpu.sync_copy(minn_v, minn_out.at[wid])


@functools.cache
def _build_sc_prop():
    return functools.partial(
        pl.kernel,
        out_type=jax.ShapeDtypeStruct((NC, NS, ROWS_PT, D), jnp.float32),
        mesh=_mesh(),
        compiler_params=pltpu.CompilerParams(needs_layout_passes=False),
        scratch_types=[
            pltpu.VMEM_SHARED((N, D), jnp.float32),
            pltpu.VMEM((ZROWS, D), jnp.float32),
        ] + [pltpu.VMEM((CHUNK, D), jnp.float32) for _ in range(NB)]
          + [pltpu.VMEM((CHUNK,), jnp.int32) for _ in range(2 * NB)]
          + [pltpu.SemaphoreType.DMA for _ in range(4 * NB)],
    )(_sc_prop_body)


def _sc_prop_body(ht_hbm, src4, dst4, s_out, acc, zero_v, *bufs):
    rows = bufs[:NB]
    src_i = bufs[NB:2 * NB]
    dst_i = bufs[2 * NB:3 * NB]
    sem_g = bufs[3 * NB:4 * NB]
    sem_s = bufs[4 * NB:5 * NB]
    sem_is = bufs[5 * NB:6 * NB]
    sem_id = bufs[6 * NB:7 * NB]
    c = lax.axis_index("c")
    s = lax.axis_index("s")
    wid = c * NS + s

    zf = jnp.zeros((16,), jnp.float32)

    def zero_buf(i, _):
        zero_v[i // 8, pl.ds((i % 8) * 16, 16)] = zf
        return ()
    lax.fori_loop(0, ZROWS * (D // 16), zero_buf, ())

    def zero_acc(i, _):
        pltpu.sync_copy(zero_v, acc.at[pl.ds(s * ROWS_PT + i * ZROWS, ZROWS)])
        return ()
    lax.fori_loop(0, ROWS_PT // ZROWS, zero_acc, ())

    plsc.subcore_barrier()

    for b in range(NB):
        pltpu.async_copy(src4.at[wid, b, 0], src_i[b], sem_is[b])
        pltpu.async_copy(dst4.at[wid, b, 0], dst_i[b], sem_id[b])

    def outer(k, _):
        for b in range(NB):
            i = k * NB + b

            @pl.when(k > 0)
            def _free_ring():
                pltpu.make_async_copy(rows[b], acc.at[dst_i[b]],
                                      sem_s[b]).wait()
                pltpu.async_copy(dst4.at[wid, i, 0], dst_i[b], sem_id[b])
            pltpu.make_async_copy(src4.at[wid, i, 0], src_i[b],
                                  sem_is[b]).wait()
            pltpu.async_copy(ht_hbm.at[src_i[b]], rows[b], sem_g[b])
        for b in range(NB):
            i = k * NB + b
            pltpu.make_async_copy(ht_hbm.at[src_i[b]], rows[b],
                                  sem_g[b]).wait()

            @pl.when(i + NB < NCHUNK)
            def _prefetch_src():
                pltpu.async_copy(src4.at[wid, i + NB, 0], src_i[b], sem_is[b])
            pltpu.make_async_copy(dst4.at[wid, i, 0], dst_i[b],
                                  sem_id[b]).wait()
            pltpu.async_copy(rows[b], acc.at[dst_i[b]], sem_s[b], add=True)
        return ()
    lax.fori_loop(0, NGRP, outer, ())

    for t in range(NGRP * NB, NCHUNK):
        b = t % NB
        pltpu.make_async_copy(rows[b], acc.at[dst_i[b]], sem_s[b]).wait()
        pltpu.async_copy(dst4.at[wid, t, 0], dst_i[b], sem_id[b])
        pltpu.make_async_copy(src4.at[wid, t, 0], src_i[b], sem_is[b]).wait()
        pltpu.async_copy(ht_hbm.at[src_i[b]], rows[b], sem_g[b])
        pltpu.make_async_copy(ht_hbm.at[src_i[b]], rows[b], sem_g[b]).wait()
        pltpu.make_async_copy(dst4.at[wid, t, 0], dst_i[b], sem_id[b]).wait()
        pltpu.async_copy(rows[b], acc.at[dst_i[b]], sem_s[b], add=True)

    for b in range(NB):
        pltpu.make_async_copy(rows[b], acc.at[dst_i[b]], sem_s[b]).wait()

    plsc.subcore_barrier()

    pltpu.sync_copy(acc.at[pl.ds(s * ROWS_PT, ROWS_PT)], s_out.at[c, s])


RB = 400
NBLK = N // RB


def _tc0_body(x_ref, w_ref, degp_ref, ht_ref, dinv_ref):
    deg = jnp.sum(degp_ref[...][:, 0, 0, :], axis=0) + 1.0
    dinv = lax.rsqrt(deg)
    y = jnp.dot(x_ref[...], w_ref[...], preferred_element_type=jnp.float32)
    ht_ref[...] = y * dinv[:, None]
    dinv_ref[...] = dinv[:, None]


def _tc0(x, w0, deg_part):
    return pl.pallas_call(
        _tc0_body,
        grid=(NBLK,),
        in_specs=[
            pl.BlockSpec((RB, D), lambda i: (i, 0)),
            pl.BlockSpec((D, D), lambda i: (0, 0)),
            pl.BlockSpec((NW, 1, 1, RB), lambda i: (0, i, 0, 0)),
        ],
        out_specs=[
            pl.BlockSpec((RB, D), lambda i: (i, 0)),
            pl.BlockSpec((RB, 1), lambda i: (i, 0)),
        ],
        out_shape=[
            jax.ShapeDtypeStruct((N, D), jnp.float32),
            jax.ShapeDtypeStruct((N, 1), jnp.float32),
        ],
    )(x, w0, deg_part.reshape(NW, NBLK, 1, RB))


def _tc_mid_body(s_ref, ht_ref, dinv_ref, b_ref, g_ref, be_ref, w_ref, o_ref):
    dinv = dinv_ref[...]
    sm = s_ref[...]
    z = dinv * (sm[0] + sm[1] + ht_ref[...]) + b_ref[...]
    a = jax.nn.relu(z * BN_C * g_ref[...] + be_ref[...])
    y = jnp.dot(a, w_ref[...], preferred_element_type=jnp.float32)
    o_ref[...] = y * dinv


def _tc_mid(s_part, ht, dinv, b, gamma, beta, w_next):
    return pl.pallas_call(
        _tc_mid_body,
        grid=(NBLK,),
        in_specs=[
            pl.BlockSpec((NC, RB, D), lambda i: (0, i, 0)),
            pl.BlockSpec((RB, D), lambda i: (i, 0)),
            pl.BlockSpec((RB, 1), lambda i: (i, 0)),
            pl.BlockSpec((1, D), lambda i: (0, 0)),
            pl.BlockSpec((1, D), lambda i: (0, 0)),
            pl.BlockSpec((1, D), lambda i: (0, 0)),
            pl.BlockSpec((D, D), lambda i: (0, 0)),
        ],
        out_specs=pl.BlockSpec((RB, D), lambda i: (i, 0)),
        out_shape=jax.ShapeDtypeStruct((N, D), jnp.float32),
    )(s_part, ht, dinv, b.reshape(1, D), gamma.reshape(1, D),
      beta.reshape(1, D), w_next)


GB = RB // P


def _tc_fin_body(s_ref, ht_ref, dinv_ref, b_ref, flag_ref, minn_ref, o_ref):
    i = pl.program_id(0)
    sm = s_ref[...]
    h3 = dinv_ref[...] * (sm[0] + sm[1] + ht_ref[...]) + b_ref[...]
    ga = lax.broadcasted_iota(jnp.int32, (GB, RB), 0)
    ra = lax.broadcasted_iota(jnp.int32, (GB, RB), 1) // P
    sel = jnp.where(ga == ra, 1.0 / P, 0.0).astype(jnp.float32)
    pooled = jnp.dot(sel, h3, preferred_element_type=jnp.float32)
    flg = jnp.max(flag_ref[...][:, 0], axis=0)
    mn = jnp.min(jnp.min(minn_ref[...][:, 0], axis=2), axis=0)
    aa = lax.broadcasted_iota(jnp.int32, (GB, D), 0)
    jj = lax.broadcasted_iota(jnp.int32, (GB, D), 1)
    nid = ((i * GB + aa) * P + jj).astype(jnp.float32)
    mask = (flg > 0.5) & (nid != mn[:, None])
    outv = jnp.where(mask, jnp.float32(-1e10), pooled)
    o_ref[...] = outv[None, :, :D_OUT]


def _tc_fin(s_part, ht, dinv, b2p, flag_part, minn_part):
    return pl.pallas_call(
        _tc_fin_body,
        grid=(NBLK,),
        in_specs=[
            pl.BlockSpec((NC, RB, D), lambda i: (0, i, 0)),
            pl.BlockSpec((RB, D), lambda i: (i, 0)),
            pl.BlockSpec((RB, 1), lambda i: (i, 0)),
            pl.BlockSpec((1, D), lambda i: (0, 0)),
            pl.BlockSpec((NW, 1, GB, D), lambda i: (0, i, 0, 0)),
            pl.BlockSpec((NW, 1, GB, 16), lambda i: (0, i, 0, 0)),
        ],
        out_specs=pl.BlockSpec((1, GB, D_OUT), lambda i: (i, 0, 0)),
        out_shape=jax.ShapeDtypeStruct((NBLK, GB, D_OUT), jnp.float32),
    )(s_part, ht, dinv, b2p.reshape(1, D),
      flag_part.reshape(NW, NBLK, GB, D),
      minn_part.reshape(NW, NBLK, GB, 16)).reshape(G, D_OUT)


def kernel(x, edge_index, batch, W0, b0, gamma0, beta0,
           W1, b1, gamma1, beta1, W2, b2):
    src = edge_index[0]
    dst = edge_index[1]

    sc_pre = _build_sc_pre()
    sc_prop = _build_sc_prop()
    deg_part, flag_part, minn_part = sc_pre(src, dst)
    deg_part = deg_part.reshape(NW, N)
    src2d = src.reshape(NW, NCHUNK, 1, CHUNK)
    dst2d = dst.reshape(NW, NCHUNK, 1, CHUNK)

    ht0, dinv = _tc0(x, W0, deg_part)
    s0 = sc_prop(ht0, src2d, dst2d).reshape(NC, N, D)
    ht1 = _tc_mid(s0, ht0, dinv, b0, gamma0, beta0, W1)
    s1 = sc_prop(ht1, src2d, dst2d).reshape(NC, N, D)
    w2p = jnp.pad(W2, ((0, 0), (0, D - D_OUT)))
    ht2 = _tc_mid(s1, ht1, dinv, b1, gamma1, beta1, w2p)
    s2 = sc_prop(ht2, src2d, dst2d).reshape(NC, N, D)
    b2p = jnp.pad(b2, (0, D - D_OUT))
    return _tc_fin(s2, ht2, dinv, b2p, flag_part, minn_part)


# speedup vs baseline: 3.2585x; 1.0197x over previous
import functools

import jax
import jax.numpy as jnp
from jax import lax
from jax.experimental import pallas as pl
from jax.experimental.pallas import tpu as pltpu
from jax.experimental.pallas import tpu_sc as plsc

N = 10000
E = 320000
G = 100
P = 100
D = 128
D_OUT = 100

NC = 2
NS = 16
NW = NC * NS
EPW = E // NW
CHUNK = 80
NB = 4
NCHUNK = EPW // CHUNK
NGRP = (NCHUNK - 1) // NB
ROWS_PT = N // NS
ZROWS = 25
IDXB = 2000
BN_C = 1.0 / (1.0 + 1e-5) ** 0.5

def _mesh():
    return plsc.VectorSubcoreMesh(core_axis_name="c", subcore_axis_name="s",
                                  num_cores=NC, num_subcores=NS)


@functools.cache
def _build_sc_pre():
    return functools.partial(
        pl.kernel,
        out_type=(
            jax.ShapeDtypeStruct((NW, 1, N), jnp.float32),
            jax.ShapeDtypeStruct((NW, G, D), jnp.float32),
            jax.ShapeDtypeStruct((NW, G, 16), jnp.float32),
        ),
        mesh=_mesh(),
        compiler_params=pltpu.CompilerParams(needs_layout_passes=False),
        scratch_types=[
            pltpu.VMEM((N,), jnp.float32),
            pltpu.VMEM((G, D), jnp.float32),
            pltpu.VMEM((G, 16), jnp.float32),
            pltpu.VMEM((EPW,), jnp.int32),
            pltpu.VMEM((EPW,), jnp.int32),
            pltpu.SemaphoreType.DMA,
        ],
    )(_sc_pre_body)


def _sc_pre_body(src_hbm, dst_hbm, deg_out, flag_out, minn_out,
                 deg_v, flag_v, minn_v, src_b, dst_b, sem):
    c = lax.axis_index("c")
    s = lax.axis_index("s")
    wid = s * NC + c
    base = wid * EPW

    pltpu.async_copy(src_hbm.at[pl.ds(base, EPW)], src_b, sem)
    pltpu.async_copy(dst_hbm.at[pl.ds(base, EPW)], dst_b, sem)

    zf = jnp.zeros((16,), jnp.float32)

    def zero_deg(i, _):
        for u in range(5):
            deg_v[pl.ds((i * 5 + u) * 16, 16)] = zf
        return ()
    lax.fori_loop(0, N // 80, zero_deg, ())

    def zero_flag(i, _):
        for u in range(8):
            flag_v[i, pl.ds(u * 16, 16)] = zf
        return ()
    lax.fori_loop(0, G, zero_flag, ())

    def init_minn(i, _):
        minn_v[i, :] = jnp.full((16,), float(N), jnp.float32)
        return ()
    lax.fori_loop(0, G, init_minn, ())

    pltpu.make_async_copy(src_hbm.at[pl.ds(base, EPW)], src_b, sem).wait()
    pltpu.make_async_copy(dst_hbm.at[pl.ds(base, EPW)], dst_b, sem).wait()

    lane = lax.iota(jnp.int32, 16)
    ones = jnp.ones((16,), jnp.float32)

    def inner(j, _):
        for u in range(5):
            jj = j * 5 + u
            src16 = src_b[pl.ds(jj * 16, 16)]
            dst16 = dst_b[pl.ds(jj * 16, 16)]
            plsc.addupdate_scatter(deg_v, [dst16], ones)
            g_src = src16 // P
            p_src = src16 % P
            g_dst = dst16 // P
            p_dst = dst16 % P
            plsc.store_scatter(flag_v, [g_src, p_src], ones)
            same = g_src == g_dst
            plsc.store_scatter(flag_v, [g_dst, p_dst], ones, mask=same)
            cur = plsc.load_gather(minn_v, [g_src, lane])
            cand = jnp.minimum(src16, dst16).astype(jnp.float32)
            plsc.store_scatter(minn_v, [g_src, lane], jnp.minimum(cur, cand))
        return ()
    lax.fori_loop(0, EPW // 80, inner, ())

    pltpu.sync_copy(deg_v, deg_out.at[wid, 0])
    pltpu.sync_copy(flag_v, flag_out.at[wid])
    pltpu.sync_copy(minn_v, minn_out.at[wid])


@functools.cache
def _build_sc_prop():
    return functools.partial(
        pl.kernel,
        out_type=jax.ShapeDtypeStruct((NC, NS, ROWS_PT, D), jnp.float32),
        mesh=_mesh(),
        compiler_params=pltpu.CompilerParams(needs_layout_passes=False),
        scratch_types=[
            pltpu.VMEM_SHARED((N, D), jnp.float32),
            pltpu.VMEM((ZROWS, D), jnp.float32),
        ] + [pltpu.VMEM((CHUNK, D), jnp.float32) for _ in range(NB)]
          + [pltpu.VMEM((CHUNK,), jnp.int32) for _ in range(2 * NB)]
          + [pltpu.SemaphoreType.DMA for _ in range(4 * NB)],
    )(_sc_prop_body)


def _sc_prop_body(ht_hbm, src4, dst4, s_out, acc, zero_v, *bufs):
    rows = bufs[:NB]
    src_i = bufs[NB:2 * NB]
    dst_i = bufs[2 * NB:3 * NB]
    sem_g = bufs[3 * NB:4 * NB]
    sem_s = bufs[4 * NB:5 * NB]
    sem_is = bufs[5 * NB:6 * NB]
    sem_id = bufs[6 * NB:7 * NB]
    c = lax.axis_index("c")
    s = lax.axis_index("s")
    wid = c * NS + s

    zf = jnp.zeros((16,), jnp.float32)

    def zero_buf(i, _):
        zero_v[i // 8, pl.ds((i % 8) * 16, 16)] = zf
        return ()
    lax.fori_loop(0, ZROWS * (D // 16), zero_buf, ())

    def zero_acc(i, _):
        pltpu.sync_copy(zero_v, acc.at[pl.ds(s * ROWS_PT + i * ZROWS, ZROWS)])
        return ()
    lax.fori_loop(0, ROWS_PT // ZROWS, zero_acc, ())

    plsc.subcore_barrier()

    for b in range(NB):
        pltpu.async_copy(src4.at[wid, b, 0], src_i[b], sem_is[b])
        pltpu.async_copy(dst4.at[wid, b, 0], dst_i[b], sem_id[b])

    def outer(k, _):
        for b in range(NB):
            i = k * NB + b

            @pl.when(k > 0)
            def _free_ring():
                pltpu.make_async_copy(rows[b], acc.at[dst_i[b]],
                                      sem_s[b]).wait()
                pltpu.async_copy(dst4.at[wid, i, 0], dst_i[b], sem_id[b])
            pltpu.make_async_copy(src4.at[wid, i, 0], src_i[b],
                                  sem_is[b]).wait()
            pltpu.async_copy(ht_hbm.at[src_i[b]], rows[b], sem_g[b])
        for b in range(NB):
            i = k * NB + b
            pltpu.make_async_copy(ht_hbm.at[src_i[b]], rows[b],
                                  sem_g[b]).wait()

            @pl.when(i + NB < NCHUNK)
            def _prefetch_src():
                pltpu.async_copy(src4.at[wid, i + NB, 0], src_i[b], sem_is[b])
            pltpu.make_async_copy(dst4.at[wid, i, 0], dst_i[b],
                                  sem_id[b]).wait()
            pltpu.async_copy(rows[b], acc.at[dst_i[b]], sem_s[b], add=True)
        return ()
    lax.fori_loop(0, NGRP, outer, ())

    for t in range(NGRP * NB, NCHUNK):
        b = t % NB
        pltpu.make_async_copy(rows[b], acc.at[dst_i[b]], sem_s[b]).wait()
        pltpu.async_copy(dst4.at[wid, t, 0], dst_i[b], sem_id[b])
        pltpu.make_async_copy(src4.at[wid, t, 0], src_i[b], sem_is[b]).wait()
        pltpu.async_copy(ht_hbm.at[src_i[b]], rows[b], sem_g[b])
        pltpu.make_async_copy(ht_hbm.at[src_i[b]], rows[b], sem_g[b]).wait()
        pltpu.make_async_copy(dst4.at[wid, t, 0], dst_i[b], sem_id[b]).wait()
        pltpu.async_copy(rows[b], acc.at[dst_i[b]], sem_s[b], add=True)

    for b in range(NB):
        pltpu.make_async_copy(rows[b], acc.at[dst_i[b]], sem_s[b]).wait()

    plsc.subcore_barrier()

    pltpu.sync_copy(acc.at[pl.ds(s * ROWS_PT, ROWS_PT)], s_out.at[c, s])


RB = 400
NBLK = N // RB


def _tc0_body(x_ref, w_ref, degp_ref, ht_ref, dinv_ref):
    deg = jnp.sum(degp_ref[...][:, 0, 0, :], axis=0) + 1.0
    dinv = lax.rsqrt(deg)
    y = jnp.dot(x_ref[...], w_ref[...], preferred_element_type=jnp.float32)
    ht_ref[...] = y * dinv[:, None]
    dinv_ref[...] = dinv[:, None]


def _tc0(x, w0, deg_part):
    return pl.pallas_call(
        _tc0_body,
        grid=(NBLK,),
        in_specs=[
            pl.BlockSpec((RB, D), lambda i: (i, 0)),
            pl.BlockSpec((D, D), lambda i: (0, 0)),
            pl.BlockSpec((NW, 1, 1, RB), lambda i: (0, i, 0, 0)),
        ],
        out_specs=[
            pl.BlockSpec((RB, D), lambda i: (i, 0)),
            pl.BlockSpec((RB, 1), lambda i: (i, 0)),
        ],
        out_shape=[
            jax.ShapeDtypeStruct((N, D), jnp.float32),
            jax.ShapeDtypeStruct((N, 1), jnp.float32),
        ],
    )(x, w0, deg_part.reshape(NW, NBLK, 1, RB))


def _tc_mid_body(s_ref, ht_ref, dinv_ref, b_ref, g_ref, be_ref, w_ref, o_ref):
    dinv = dinv_ref[...]
    sm = s_ref[...]
    z = dinv * (sm[0] + sm[1] + ht_ref[...]) + b_ref[...]
    a = jax.nn.relu(z * BN_C * g_ref[...] + be_ref[...])
    y = jnp.dot(a, w_ref[...], preferred_element_type=jnp.float32)
    o_ref[...] = y * dinv


def _tc_mid(s_part, ht, dinv, b, gamma, beta, w_next):
    return pl.pallas_call(
        _tc_mid_body,
        grid=(NBLK,),
        in_specs=[
            pl.BlockSpec((NC, RB, D), lambda i: (0, i, 0)),
            pl.BlockSpec((RB, D), lambda i: (i, 0)),
            pl.BlockSpec((RB, 1), lambda i: (i, 0)),
            pl.BlockSpec((1, D), lambda i: (0, 0)),
            pl.BlockSpec((1, D), lambda i: (0, 0)),
            pl.BlockSpec((1, D), lambda i: (0, 0)),
            pl.BlockSpec((D, D), lambda i: (0, 0)),
        ],
        out_specs=pl.BlockSpec((RB, D), lambda i: (i, 0)),
        out_shape=jax.ShapeDtypeStruct((N, D), jnp.float32),
    )(s_part, ht, dinv, b.reshape(1, D), gamma.reshape(1, D),
      beta.reshape(1, D), w_next)


GB = RB // P


def _tc_fin_body(s_ref, ht_ref, dinv_ref, b_ref, flag_ref, minn_ref, o_ref):
    i = pl.program_id(0)
    sm = s_ref[...]
    h3 = dinv_ref[...] * (sm[0] + sm[1] + ht_ref[...]) + b_ref[...]
    ga = lax.broadcasted_iota(jnp.int32, (GB, RB), 0)
    ra = lax.broadcasted_iota(jnp.int32, (GB, RB), 1) // P
    sel = jnp.where(ga == ra, 1.0 / P, 0.0).astype(jnp.float32)
    pooled = jnp.dot(sel, h3, preferred_element_type=jnp.float32)
    flg = jnp.max(flag_ref[...][:, 0], axis=0)
    mn = jnp.min(jnp.min(minn_ref[...][:, 0], axis=2), axis=0)
    aa = lax.broadcasted_iota(jnp.int32, (GB, D), 0)
    jj = lax.broadcasted_iota(jnp.int32, (GB, D), 1)
    nid = ((i * GB + aa) * P + jj).astype(jnp.float32)
    mask = (flg > 0.5) & (nid != mn[:, None])
    outv = jnp.where(mask, jnp.float32(-1e10), pooled)
    o_ref[...] = outv[None, :, :D_OUT]


def _tc_fin(s_part, ht, dinv, b2p, flag_part, minn_part):
    return pl.pallas_call(
        _tc_fin_body,
        grid=(NBLK,),
        in_specs=[
            pl.BlockSpec((NC, RB, D), lambda i: (0, i, 0)),
            pl.BlockSpec((RB, D), lambda i: (i, 0)),
            pl.BlockSpec((RB, 1), lambda i: (i, 0)),
            pl.BlockSpec((1, D), lambda i: (0, 0)),
            pl.BlockSpec((NW, 1, GB, D), lambda i: (0, i, 0, 0)),
            pl.BlockSpec((NW, 1, GB, 16), lambda i: (0, i, 0, 0)),
        ],
        out_specs=pl.BlockSpec((1, GB, D_OUT), lambda i: (i, 0, 0)),
        out_shape=jax.ShapeDtypeStruct((NBLK, GB, D_OUT), jnp.float32),
    )(s_part, ht, dinv, b2p.reshape(1, D),
      flag_part.reshape(NW, NBLK, GB, D),
      minn_part.reshape(NW, NBLK, GB, 16)).reshape(G, D_OUT)


def kernel(x, edge_index, batch, W0, b0, gamma0, beta0,
           W1, b1, gamma1, beta1, W2, b2):
    src = edge_index[0]
    dst = edge_index[1]

    sc_pre = _build_sc_pre()
    sc_prop = _build_sc_prop()
    deg_part, flag_part, minn_part = sc_pre(src, dst)
    deg_part = deg_part.reshape(NW, N)
    src2d = src.reshape(NW, NCHUNK, 1, CHUNK)
    dst2d = dst.reshape(NW, NCHUNK, 1, CHUNK)

    ht0, dinv = _tc0(x, W0, deg_part)
    s0 = sc_prop(ht0, src2d, dst2d).reshape(NC, N, D)
    ht1 = _tc_mid(s0, ht0, dinv, b0, gamma0, beta0, W1)
    s1 = sc_prop(ht1, src2d, dst2d).reshape(NC, N, D)
    w2p = jnp.pad(W2, ((0, 0), (0, D - D_OUT)))
    ht2 = _tc_mid(s1, ht1, dinv, b1, gamma1, beta1, w2p)
    s2 = sc_prop(ht2, src2d, dst2d).reshape(NC, N, D)
    b2p = jnp.pad(b2, (0, D - D_OUT))
    return _tc_fin(s2, ht2, dinv, b2p, flag_part, minn_part)


# speedup vs baseline: 3.2594x; 1.0003x over previous
import functools

import jax
import jax.numpy as jnp
from jax import lax
from jax.experimental import pallas as pl
from jax.experimental.pallas import tpu as pltpu
from jax.experimental.pallas import tpu_sc as plsc

N = 10000
E = 320000
G = 100
P = 100
D = 128
D_OUT = 100

NC = 2
NS = 16
NW = NC * NS
EPW = E // NW
CHUNK = 80
NB = 4
NCHUNK = EPW // CHUNK
NGRP = (NCHUNK - 1) // NB
ROWS_PT = N // NS
ZROWS = 25
IDXB = 2000
BN_C = 1.0 / (1.0 + 1e-5) ** 0.5

def _mesh():
    return plsc.VectorSubcoreMesh(core_axis_name="c", subcore_axis_name="s",
                                  num_cores=NC, num_subcores=NS)


@functools.cache
def _build_sc_pre():
    return functools.partial(
        pl.kernel,
        out_type=(
            jax.ShapeDtypeStruct((NW, 1, N), jnp.float32),
            jax.ShapeDtypeStruct((NW, G, D), jnp.float32),
            jax.ShapeDtypeStruct((NW, G, 16), jnp.float32),
        ),
        mesh=_mesh(),
        compiler_params=pltpu.CompilerParams(needs_layout_passes=False),
        scratch_types=[
            pltpu.VMEM((N,), jnp.float32),
            pltpu.VMEM((G, D), jnp.float32),
            pltpu.VMEM((G, 16), jnp.float32),
            pltpu.VMEM((EPW,), jnp.int32),
            pltpu.VMEM((EPW,), jnp.int32),
            pltpu.SemaphoreType.DMA,
        ],
    )(_sc_pre_body)


def _sc_pre_body(src_hbm, dst_hbm, deg_out, flag_out, minn_out,
                 deg_v, flag_v, minn_v, src_b, dst_b, sem):
    c = lax.axis_index("c")
    s = lax.axis_index("s")
    wid = s * NC + c
    base = wid * EPW

    pltpu.async_copy(src_hbm.at[pl.ds(base, EPW)], src_b, sem)
    pltpu.async_copy(dst_hbm.at[pl.ds(base, EPW)], dst_b, sem)

    zf = jnp.zeros((16,), jnp.float32)

    def zero_deg(i, _):
        for u in range(5):
            deg_v[pl.ds((i * 5 + u) * 16, 16)] = zf
        return ()
    lax.fori_loop(0, N // 80, zero_deg, ())

    def zero_flag(i, _):
        for u in range(8):
            flag_v[i, pl.ds(u * 16, 16)] = zf
        return ()
    lax.fori_loop(0, G, zero_flag, ())

    def init_minn(i, _):
        minn_v[i, :] = jnp.full((16,), float(N), jnp.float32)
        return ()
    lax.fori_loop(0, G, init_minn, ())

    pltpu.make_async_copy(src_hbm.at[pl.ds(base, EPW)], src_b, sem).wait()
    pltpu.make_async_copy(dst_hbm.at[pl.ds(base, EPW)], dst_b, sem).wait()

    lane = lax.iota(jnp.int32, 16)
    ones = jnp.ones((16,), jnp.float32)

    def inner(j, _):
        for u in range(5):
            jj = j * 5 + u
            src16 = src_b[pl.ds(jj * 16, 16)]
            dst16 = dst_b[pl.ds(jj * 16, 16)]
            plsc.addupdate_scatter(deg_v, [dst16], ones)
            g_src = src16 // P
            p_src = src16 % P
            g_dst = dst16 // P
            p_dst = dst16 % P
            plsc.store_scatter(flag_v, [g_src, p_src], ones)
            same = g_src == g_dst
            plsc.store_scatter(flag_v, [g_dst, p_dst], ones, mask=same)
            cur = plsc.load_gather(minn_v, [g_src, lane])
            cand = jnp.minimum(src16, dst16).astype(jnp.float32)
            plsc.store_scatter(minn_v, [g_src, lane], jnp.minimum(cur, cand))
        return ()
    lax.fori_loop(0, EPW // 80, inner, ())

    pltpu.sync_copy(deg_v, deg_out.at[wid, 0])
    pltpu.sync_copy(flag_v, flag_out.at[wid])
    pltpu.sync_copy(minn_v, minn_out.at[wid])


@functools.cache
def _build_sc_prop():
    return functools.partial(
        pl.kernel,
        out_type=jax.ShapeDtypeStruct((NC, NS, ROWS_PT, D), jnp.float32),
        mesh=_mesh(),
        compiler_params=pltpu.CompilerParams(needs_layout_passes=False),
        scratch_types=[
            pltpu.VMEM_SHARED((N, D), jnp.float32),
            pltpu.VMEM((ZROWS, D), jnp.float32),
        ] + [pltpu.VMEM((CHUNK, D), jnp.float32) for _ in range(NB)]
          + [pltpu.VMEM((CHUNK,), jnp.int32) for _ in range(2 * NB)]
          + [pltpu.SemaphoreType.DMA for _ in range(4 * NB)],
    )(_sc_prop_body)


def _sc_prop_body(ht_hbm, src4, dst4, s_out, acc, zero_v, *bufs):
    rows = bufs[:NB]
    src_i = bufs[NB:2 * NB]
    dst_i = bufs[2 * NB:3 * NB]
    sem_g = bufs[3 * NB:4 * NB]
    sem_s = bufs[4 * NB:5 * NB]
    sem_is = bufs[5 * NB:6 * NB]
    sem_id = bufs[6 * NB:7 * NB]
    c = lax.axis_index("c")
    s = lax.axis_index("s")
    wid = c * NS + s

    zf = jnp.zeros((16,), jnp.float32)

    def zero_buf(i, _):
        zero_v[i // 8, pl.ds((i % 8) * 16, 16)] = zf
        return ()
    lax.fori_loop(0, ZROWS * (D // 16), zero_buf, ())

    def zero_acc(i, _):
        pltpu.sync_copy(zero_v, acc.at[pl.ds(s * ROWS_PT + i * ZROWS, ZROWS)])
        return ()
    lax.fori_loop(0, ROWS_PT // ZROWS, zero_acc, ())

    plsc.subcore_barrier()

    for b in range(NB):
        pltpu.async_copy(src4.at[wid, b, 0], src_i[b], sem_is[b])
        pltpu.async_copy(dst4.at[wid, b, 0], dst_i[b], sem_id[b])

    def outer(k, _):
        for b in range(NB):
            i = k * NB + b

            @pl.when(k > 0)
            def _free_ring():
                pltpu.make_async_copy(rows[b], acc.at[dst_i[b]],
                                      sem_s[b]).wait()
                pltpu.async_copy(dst4.at[wid, i, 0], dst_i[b], sem_id[b])
            pltpu.make_async_copy(src4.at[wid, i, 0], src_i[b],
                                  sem_is[b]).wait()
            pltpu.async_copy(ht_hbm.at[src_i[b]], rows[b], sem_g[b])
        for b in range(NB):
            i = k * NB + b
            pltpu.make_async_copy(ht_hbm.at[src_i[b]], rows[b],
                                  sem_g[b]).wait()

            @pl.when(i + NB < NCHUNK)
            def _prefetch_src():
                pltpu.async_copy(src4.at[wid, i + NB, 0], src_i[b], sem_is[b])
            pltpu.make_async_copy(dst4.at[wid, i, 0], dst_i[b],
                                  sem_id[b]).wait()
            pltpu.async_copy(rows[b], acc.at[dst_i[b]], sem_s[b], add=True)
        return ()
    lax.fori_loop(0, NGRP, outer, ())

    for t in range(NGRP * NB, NCHUNK):
        b = t % NB
        pltpu.make_async_copy(rows[b], acc.at[dst_i[b]], sem_s[b]).wait()
        pltpu.async_copy(dst4.at[wid, t, 0], dst_i[b], sem_id[b])
        pltpu.make_async_copy(src4.at[wid, t, 0], src_i[b], sem_is[b]).wait()
        pltpu.async_copy(ht_hbm.at[src_i[b]], rows[b], sem_g[b])
        pltpu.make_async_copy(ht_hbm.at[src_i[b]], rows[b], sem_g[b]).wait()
        pltpu.make_async_copy(dst4.at[wid, t, 0], dst_i[b], sem_id[b]).wait()
        pltpu.async_copy(rows[b], acc.at[dst_i[b]], sem_s[b], add=True)

    for b in range(NB):
        pltpu.make_async_copy(rows[b], acc.at[dst_i[b]], sem_s[b]).wait()

    plsc.subcore_barrier()

    pltpu.sync_copy(acc.at[pl.ds(s * ROWS_PT, ROWS_PT)], s_out.at[c, s])


RB = 400
NBLK = N // RB


def _tc_mm_body(x_ref, w_ref, y_ref):
    y_ref[...] = jnp.dot(x_ref[...], w_ref[...],
                         preferred_element_type=jnp.float32)


def _tc_mm(x, w0):
    return pl.pallas_call(
        _tc_mm_body,
        grid=(NBLK,),
        in_specs=[
            pl.BlockSpec((RB, D), lambda i: (i, 0)),
            pl.BlockSpec((D, D), lambda i: (0, 0)),
        ],
        out_specs=pl.BlockSpec((RB, D), lambda i: (i, 0)),
        out_shape=jax.ShapeDtypeStruct((N, D), jnp.float32),
    )(x, w0)


def _tc0_body(y_ref, degp_ref, ht_ref, dinv_ref):
    deg = jnp.sum(degp_ref[...][:, 0, 0, :], axis=0) + 1.0
    dinv = lax.rsqrt(deg)
    ht_ref[...] = y_ref[...] * dinv[:, None]
    dinv_ref[...] = dinv[:, None]


def _tc0(y0, deg_part):
    return pl.pallas_call(
        _tc0_body,
        grid=(NBLK,),
        in_specs=[
            pl.BlockSpec((RB, D), lambda i: (i, 0)),
            pl.BlockSpec((NW, 1, 1, RB), lambda i: (0, i, 0, 0)),
        ],
        out_specs=[
            pl.BlockSpec((RB, D), lambda i: (i, 0)),
            pl.BlockSpec((RB, 1), lambda i: (i, 0)),
        ],
        out_shape=[
            jax.ShapeDtypeStruct((N, D), jnp.float32),
            jax.ShapeDtypeStruct((N, 1), jnp.float32),
        ],
    )(y0, deg_part.reshape(NW, NBLK, 1, RB))


def _tc_mid_body(s_ref, ht_ref, dinv_ref, b_ref, g_ref, be_ref, w_ref, o_ref):
    dinv = dinv_ref[...]
    sm = s_ref[...]
    z = dinv * (sm[0] + sm[1] + ht_ref[...]) + b_ref[...]
    a = jax.nn.relu(z * BN_C * g_ref[...] + be_ref[...])
    y = jnp.dot(a, w_ref[...], preferred_element_type=jnp.float32)
    o_ref[...] = y * dinv


def _tc_mid(s_part, ht, dinv, b, gamma, beta, w_next):
    return pl.pallas_call(
        _tc_mid_body,
        grid=(NBLK,),
        in_specs=[
            pl.BlockSpec((NC, RB, D), lambda i: (0, i, 0)),
            pl.BlockSpec((RB, D), lambda i: (i, 0)),
            pl.BlockSpec((RB, 1), lambda i: (i, 0)),
            pl.BlockSpec((1, D), lambda i: (0, 0)),
            pl.BlockSpec((1, D), lambda i: (0, 0)),
            pl.BlockSpec((1, D), lambda i: (0, 0)),
            pl.BlockSpec((D, D), lambda i: (0, 0)),
        ],
        out_specs=pl.BlockSpec((RB, D), lambda i: (i, 0)),
        out_shape=jax.ShapeDtypeStruct((N, D), jnp.float32),
    )(s_part, ht, dinv, b.reshape(1, D), gamma.reshape(1, D),
      beta.reshape(1, D), w_next)


GB = RB // P


def _tc_fin_body(s_ref, ht_ref, dinv_ref, b_ref, flag_ref, minn_ref, o_ref):
    i = pl.program_id(0)
    sm = s_ref[...]
    h3 = dinv_ref[...] * (sm[0] + sm[1] + ht_ref[...]) + b_ref[...]
    ga = lax.broadcasted_iota(jnp.int32, (GB, RB), 0)
    ra = lax.broadcasted_iota(jnp.int32, (GB, RB), 1) // P
    sel = jnp.where(ga == ra, 1.0 / P, 0.0).astype(jnp.float32)
    pooled = jnp.dot(sel, h3, preferred_element_type=jnp.float32)
    flg = jnp.max(flag_ref[...][:, 0], axis=0)
    mn = jnp.min(jnp.min(minn_ref[...][:, 0], axis=2), axis=0)
    aa = lax.broadcasted_iota(jnp.int32, (GB, D), 0)
    jj = lax.broadcasted_iota(jnp.int32, (GB, D), 1)
    nid = ((i * GB + aa) * P + jj).astype(jnp.float32)
    mask = (flg > 0.5) & (nid != mn[:, None])
    outv = jnp.where(mask, jnp.float32(-1e10), pooled)
    o_ref[...] = outv[None, :, :D_OUT]


def _tc_fin(s_part, ht, dinv, b2p, flag_part, minn_part):
    return pl.pallas_call(
        _tc_fin_body,
        grid=(NBLK,),
        in_specs=[
            pl.BlockSpec((NC, RB, D), lambda i: (0, i, 0)),
            pl.BlockSpec((RB, D), lambda i: (i, 0)),
            pl.BlockSpec((RB, 1), lambda i: (i, 0)),
            pl.BlockSpec((1, D), lambda i: (0, 0)),
            pl.BlockSpec((NW, 1, GB, D), lambda i: (0, i, 0, 0)),
            pl.BlockSpec((NW, 1, GB, 16), lambda i: (0, i, 0, 0)),
        ],
        out_specs=pl.BlockSpec((1, GB, D_OUT), lambda i: (i, 0, 0)),
        out_shape=jax.ShapeDtypeStruct((NBLK, GB, D_OUT), jnp.float32),
    )(s_part, ht, dinv, b2p.reshape(1, D),
      flag_part.reshape(NW, NBLK, GB, D),
      minn_part.reshape(NW, NBLK, GB, 16)).reshape(G, D_OUT)


def kernel(x, edge_index, batch, W0, b0, gamma0, beta0,
           W1, b1, gamma1, beta1, W2, b2):
    src = edge_index[0]
    dst = edge_index[1]

    sc_pre = _build_sc_pre()
    sc_prop = _build_sc_prop()
    deg_part, flag_part, minn_part = sc_pre(src, dst)
    deg_part = deg_part.reshape(NW, N)
    src2d = src.reshape(NW, NCHUNK, 1, CHUNK)
    dst2d = dst.reshape(NW, NCHUNK, 1, CHUNK)

    y0 = _tc_mm(x, W0)
    ht0, dinv = _tc0(y0, deg_part)
    s0 = sc_prop(ht0, src2d, dst2d).reshape(NC, N, D)
    ht1 = _tc_mid(s0, ht0, dinv, b0, gamma0, beta0, W1)
    s1 = sc_prop(ht1, src2d, dst2d).reshape(NC, N, D)
    w2p = jnp.pad(W2, ((0, 0), (0, D - D_OUT)))
    ht2 = _tc_mid(s1, ht1, dinv, b1, gamma1, beta1, w2p)
    s2 = sc_prop(ht2, src2d, dst2d).reshape(NC, N, D)
    b2p = jnp.pad(b2, (0, D - D_OUT))
    return _tc_fin(s2, ht2, dinv, b2p, flag_part, minn_part)


# speedup vs baseline: 3.4449x; 1.0569x over previous
import functools

import jax
import jax.numpy as jnp
from jax import lax
from jax.experimental import pallas as pl
from jax.experimental.pallas import tpu as pltpu
from jax.experimental.pallas import tpu_sc as plsc

N = 10000
E = 320000
G = 100
P = 100
D = 128
D_OUT = 100

NC = 2
NS = 16
NW = NC * NS
EPW = E // NW
CHUNK = 80
NB = 4
NCHUNK = EPW // CHUNK
NGRP = (NCHUNK - 1) // NB
ROWS_PT = N // NS
ZROWS = 25
IDXB = 2000
BN_C = 1.0 / (1.0 + 1e-5) ** 0.5

def _mesh():
    return plsc.VectorSubcoreMesh(core_axis_name="c", subcore_axis_name="s",
                                  num_cores=NC, num_subcores=NS)


@functools.cache
def _build_sc_pre():
    return functools.partial(
        pl.kernel,
        out_type=(
            jax.ShapeDtypeStruct((NW, 1, N), jnp.float32),
            jax.ShapeDtypeStruct((NW, G, D), jnp.float32),
            jax.ShapeDtypeStruct((NW, G, 16), jnp.float32),
        ),
        mesh=_mesh(),
        compiler_params=pltpu.CompilerParams(needs_layout_passes=False),
        scratch_types=[
            pltpu.VMEM((N,), jnp.float32),
            pltpu.VMEM((G, D), jnp.float32),
            pltpu.VMEM((G, 16), jnp.float32),
            pltpu.VMEM((EPW,), jnp.int32),
            pltpu.VMEM((EPW,), jnp.int32),
            pltpu.SemaphoreType.DMA,
        ],
    )(_sc_pre_body)


def _sc_pre_body(src_hbm, dst_hbm, deg_out, flag_out, minn_out,
                 deg_v, flag_v, minn_v, src_b, dst_b, sem):
    c = lax.axis_index("c")
    s = lax.axis_index("s")
    wid = s * NC + c
    base = wid * EPW

    pltpu.async_copy(src_hbm.at[pl.ds(base, EPW)], src_b, sem)
    pltpu.async_copy(dst_hbm.at[pl.ds(base, EPW)], dst_b, sem)

    zf = jnp.zeros((16,), jnp.float32)

    def zero_deg(i, _):
        for u in range(5):
            deg_v[pl.ds((i * 5 + u) * 16, 16)] = zf
        return ()
    lax.fori_loop(0, N // 80, zero_deg, ())

    def zero_flag(i, _):
        for u in range(8):
            flag_v[i, pl.ds(u * 16, 16)] = zf
        return ()
    lax.fori_loop(0, G, zero_flag, ())

    def init_minn(i, _):
        minn_v[i, :] = jnp.full((16,), float(N), jnp.float32)
        return ()
    lax.fori_loop(0, G, init_minn, ())

    pltpu.make_async_copy(src_hbm.at[pl.ds(base, EPW)], src_b, sem).wait()
    pltpu.make_async_copy(dst_hbm.at[pl.ds(base, EPW)], dst_b, sem).wait()

    lane = lax.iota(jnp.int32, 16)
    ones = jnp.ones((16,), jnp.float32)

    def inner(j, _):
        for u in range(5):
            jj = j * 5 + u
            src16 = src_b[pl.ds(jj * 16, 16)]
            dst16 = dst_b[pl.ds(jj * 16, 16)]
            plsc.addupdate_scatter(deg_v, [dst16], ones)
            g_src = (src16.astype(jnp.float32) * (1.0 / P)
                     + 0.005).astype(jnp.int32)
            p_src = src16 - g_src * P
            g_dst = (dst16.astype(jnp.float32) * (1.0 / P)
                     + 0.005).astype(jnp.int32)
            p_dst = dst16 - g_dst * P
            plsc.store_scatter(flag_v, [g_src, p_src], ones)
            same = g_src == g_dst
            plsc.store_scatter(flag_v, [g_dst, p_dst], ones, mask=same)
            cur = plsc.load_gather(minn_v, [g_src, lane])
            cand = jnp.minimum(src16, dst16).astype(jnp.float32)
            plsc.store_scatter(minn_v, [g_src, lane], jnp.minimum(cur, cand))
        return ()
    lax.fori_loop(0, EPW // 80, inner, ())

    pltpu.sync_copy(deg_v, deg_out.at[wid, 0])
    pltpu.sync_copy(flag_v, flag_out.at[wid])
    pltpu.sync_copy(minn_v, minn_out.at[wid])


@functools.cache
def _build_sc_prop():
    return functools.partial(
        pl.kernel,
        out_type=jax.ShapeDtypeStruct((NC, NS, ROWS_PT, D), jnp.float32),
        mesh=_mesh(),
        compiler_params=pltpu.CompilerParams(needs_layout_passes=False),
        scratch_types=[
            pltpu.VMEM_SHARED((N, D), jnp.float32),
            pltpu.VMEM((ZROWS, D), jnp.float32),
        ] + [pltpu.VMEM((CHUNK, D), jnp.float32) for _ in range(NB)]
          + [pltpu.VMEM((CHUNK,), jnp.int32) for _ in range(2 * NB)]
          + [pltpu.SemaphoreType.DMA for _ in range(4 * NB)],
    )(_sc_prop_body)


def _sc_prop_body(ht_hbm, src4, dst4, s_out, acc, zero_v, *bufs):
    rows = bufs[:NB]
    src_i = bufs[NB:2 * NB]
    dst_i = bufs[2 * NB:3 * NB]
    sem_g = bufs[3 * NB:4 * NB]
    sem_s = bufs[4 * NB:5 * NB]
    sem_is = bufs[5 * NB:6 * NB]
    sem_id = bufs[6 * NB:7 * NB]
    c = lax.axis_index("c")
    s = lax.axis_index("s")
    wid = c * NS + s

    zf = jnp.zeros((16,), jnp.float32)

    def zero_buf(i, _):
        zero_v[i // 8, pl.ds((i % 8) * 16, 16)] = zf
        return ()
    lax.fori_loop(0, ZROWS * (D // 16), zero_buf, ())

    def zero_acc(i, _):
        pltpu.sync_copy(zero_v, acc.at[pl.ds(s * ROWS_PT + i * ZROWS, ZROWS)])
        return ()
    lax.fori_loop(0, ROWS_PT // ZROWS, zero_acc, ())

    plsc.subcore_barrier()

    for b in range(NB):
        pltpu.async_copy(src4.at[wid, b, 0], src_i[b], sem_is[b])
        pltpu.async_copy(dst4.at[wid, b, 0], dst_i[b], sem_id[b])

    def outer(k, _):
        for b in range(NB):
            i = k * NB + b

            @pl.when(k > 0)
            def _free_ring():
                pltpu.make_async_copy(rows[b], acc.at[dst_i[b]],
                                      sem_s[b]).wait()
                pltpu.async_copy(dst4.at[wid, i, 0], dst_i[b], sem_id[b])
            pltpu.make_async_copy(src4.at[wid, i, 0], src_i[b],
                                  sem_is[b]).wait()
            pltpu.async_copy(ht_hbm.at[src_i[b]], rows[b], sem_g[b])
        for b in range(NB):
            i = k * NB + b
            pltpu.make_async_copy(ht_hbm.at[src_i[b]], rows[b],
                                  sem_g[b]).wait()

            @pl.when(i + NB < NCHUNK)
            def _prefetch_src():
                pltpu.async_copy(src4.at[wid, i + NB, 0], src_i[b], sem_is[b])
            pltpu.make_async_copy(dst4.at[wid, i, 0], dst_i[b],
                                  sem_id[b]).wait()
            pltpu.async_copy(rows[b], acc.at[dst_i[b]], sem_s[b], add=True)
        return ()
    lax.fori_loop(0, NGRP, outer, ())

    for t in range(NGRP * NB, NCHUNK):
        b = t % NB
        pltpu.make_async_copy(rows[b], acc.at[dst_i[b]], sem_s[b]).wait()
        pltpu.async_copy(dst4.at[wid, t, 0], dst_i[b], sem_id[b])
        pltpu.make_async_copy(src4.at[wid, t, 0], src_i[b], sem_is[b]).wait()
        pltpu.async_copy(ht_hbm.at[src_i[b]], rows[b], sem_g[b])
        pltpu.make_async_copy(ht_hbm.at[src_i[b]], rows[b], sem_g[b]).wait()
        pltpu.make_async_copy(dst4.at[wid, t, 0], dst_i[b], sem_id[b]).wait()
        pltpu.async_copy(rows[b], acc.at[dst_i[b]], sem_s[b], add=True)

    for b in range(NB):
        pltpu.make_async_copy(rows[b], acc.at[dst_i[b]], sem_s[b]).wait()

    plsc.subcore_barrier()

    pltpu.sync_copy(acc.at[pl.ds(s * ROWS_PT, ROWS_PT)], s_out.at[c, s])


RB = 400
NBLK = N // RB


def _tc_mm_body(x_ref, w_ref, y_ref):
    y_ref[...] = jnp.dot(x_ref[...], w_ref[...],
                         preferred_element_type=jnp.float32)


def _tc_mm(x, w0):
    return pl.pallas_call(
        _tc_mm_body,
        grid=(NBLK,),
        in_specs=[
            pl.BlockSpec((RB, D), lambda i: (i, 0)),
            pl.BlockSpec((D, D), lambda i: (0, 0)),
        ],
        out_specs=pl.BlockSpec((RB, D), lambda i: (i, 0)),
        out_shape=jax.ShapeDtypeStruct((N, D), jnp.float32),
    )(x, w0)


def _tc0_body(y_ref, degp_ref, ht_ref, dinv_ref):
    deg = jnp.sum(degp_ref[...][:, 0, 0, :], axis=0) + 1.0
    dinv = lax.rsqrt(deg)
    ht_ref[...] = y_ref[...] * dinv[:, None]
    dinv_ref[...] = dinv[:, None]


def _tc0(y0, deg_part):
    return pl.pallas_call(
        _tc0_body,
        grid=(NBLK,),
        in_specs=[
            pl.BlockSpec((RB, D), lambda i: (i, 0)),
            pl.BlockSpec((NW, 1, 1, RB), lambda i: (0, i, 0, 0)),
        ],
        out_specs=[
            pl.BlockSpec((RB, D), lambda i: (i, 0)),
            pl.BlockSpec((RB, 1), lambda i: (i, 0)),
        ],
        out_shape=[
            jax.ShapeDtypeStruct((N, D), jnp.float32),
            jax.ShapeDtypeStruct((N, 1), jnp.float32),
        ],
    )(y0, deg_part.reshape(NW, NBLK, 1, RB))


def _tc_mid_body(s_ref, ht_ref, dinv_ref, b_ref, g_ref, be_ref, w_ref, o_ref):
    dinv = dinv_ref[...]
    sm = s_ref[...]
    z = dinv * (sm[0] + sm[1] + ht_ref[...]) + b_ref[...]
    a = jax.nn.relu(z * BN_C * g_ref[...] + be_ref[...])
    y = jnp.dot(a, w_ref[...], preferred_element_type=jnp.float32)
    o_ref[...] = y * dinv


def _tc_mid(s_part, ht, dinv, b, gamma, beta, w_next):
    return pl.pallas_call(
        _tc_mid_body,
        grid=(NBLK,),
        in_specs=[
            pl.BlockSpec((NC, RB, D), lambda i: (0, i, 0)),
            pl.BlockSpec((RB, D), lambda i: (i, 0)),
            pl.BlockSpec((RB, 1), lambda i: (i, 0)),
            pl.BlockSpec((1, D), lambda i: (0, 0)),
            pl.BlockSpec((1, D), lambda i: (0, 0)),
            pl.BlockSpec((1, D), lambda i: (0, 0)),
            pl.BlockSpec((D, D), lambda i: (0, 0)),
        ],
        out_specs=pl.BlockSpec((RB, D), lambda i: (i, 0)),
        out_shape=jax.ShapeDtypeStruct((N, D), jnp.float32),
    )(s_part, ht, dinv, b.reshape(1, D), gamma.reshape(1, D),
      beta.reshape(1, D), w_next)


GB = RB // P


def _tc_fin_body(s_ref, ht_ref, dinv_ref, b_ref, flag_ref, minn_ref, o_ref):
    i = pl.program_id(0)
    sm = s_ref[...]
    h3 = dinv_ref[...] * (sm[0] + sm[1] + ht_ref[...]) + b_ref[...]
    ga = lax.broadcasted_iota(jnp.int32, (GB, RB), 0)
    ra = lax.broadcasted_iota(jnp.int32, (GB, RB), 1) // P
    sel = jnp.where(ga == ra, 1.0 / P, 0.0).astype(jnp.float32)
    pooled = jnp.dot(sel, h3, preferred_element_type=jnp.float32)
    flg = jnp.max(flag_ref[...][:, 0], axis=0)
    mn = jnp.min(jnp.min(minn_ref[...][:, 0], axis=2), axis=0)
    aa = lax.broadcasted_iota(jnp.int32, (GB, D), 0)
    jj = lax.broadcasted_iota(jnp.int32, (GB, D), 1)
    nid = ((i * GB + aa) * P + jj).astype(jnp.float32)
    mask = (flg > 0.5) & (nid != mn[:, None])
    outv = jnp.where(mask, jnp.float32(-1e10), pooled)
    o_ref[...] = outv[None, :, :D_OUT]


def _tc_fin(s_part, ht, dinv, b2p, flag_part, minn_part):
    return pl.pallas_call(
        _tc_fin_body,
        grid=(NBLK,),
        in_specs=[
            pl.BlockSpec((NC, RB, D), lambda i: (0, i, 0)),
            pl.BlockSpec((RB, D), lambda i: (i, 0)),
            pl.BlockSpec((RB, 1), lambda i: (i, 0)),
            pl.BlockSpec((1, D), lambda i: (0, 0)),
            pl.BlockSpec((NW, 1, GB, D), lambda i: (0, i, 0, 0)),
            pl.BlockSpec((NW, 1, GB, 16), lambda i: (0, i, 0, 0)),
        ],
        out_specs=pl.BlockSpec((1, GB, D_OUT), lambda i: (i, 0, 0)),
        out_shape=jax.ShapeDtypeStruct((NBLK, GB, D_OUT), jnp.float32),
    )(s_part, ht, dinv, b2p.reshape(1, D),
      flag_part.reshape(NW, NBLK, GB, D),
      minn_part.reshape(NW, NBLK, GB, 16)).reshape(G, D_OUT)


def kernel(x, edge_index, batch, W0, b0, gamma0, beta0,
           W1, b1, gamma1, beta1, W2, b2):
    src = edge_index[0]
    dst = edge_index[1]

    sc_pre = _build_sc_pre()
    sc_prop = _build_sc_prop()
    deg_part, flag_part, minn_part = sc_pre(src, dst)
    deg_part = deg_part.reshape(NW, N)
    src2d = src.reshape(NW, NCHUNK, 1, CHUNK)
    dst2d = dst.reshape(NW, NCHUNK, 1, CHUNK)

    y0 = _tc_mm(x, W0)
    ht0, dinv = _tc0(y0, deg_part)
    s0 = sc_prop(ht0, src2d, dst2d).reshape(NC, N, D)
    ht1 = _tc_mid(s0, ht0, dinv, b0, gamma0, beta0, W1)
    s1 = sc_prop(ht1, src2d, dst2d).reshape(NC, N, D)
    w2p = jnp.pad(W2, ((0, 0), (0, D - D_OUT)))
    ht2 = _tc_mid(s1, ht1, dinv, b1, gamma1, beta1, w2p)
    s2 = sc_prop(ht2, src2d, dst2d).reshape(NC, N, D)
    b2p = jnp.pad(b2, (0, D - D_OUT))
    return _tc_fin(s2, ht2, dinv, b2p, flag_part, minn_part)
